# baseline (speedup 1.0000x reference)
import functools

import jax
import jax.numpy as jnp
from jax import lax
from jax.experimental import pallas as pl
from jax.experimental.pallas import tpu as pltpu
from jax.experimental.pallas import tpu_sc as plsc

_N = 10000
_E = 320000
_IN_F = 128
_EPS = 1e-5
_NP = 10240
_W = 128

_f32 = jnp.float32

_info = plsc.get_sparse_core_info()
_NC = _info.num_cores
_NS = _info.num_subcores
_NW = _NC * _NS
_CH = 80
_EW = _E // _NW
_EC = _E // _NS
_EH = _E // 2 // _NS
_NROW = _NP // _NS

_mesh = plsc.VectorSubcoreMesh(core_axis_name="c", subcore_axis_name="s")


def _tab_body(h_ref, gs_ref, gd_ref, s_ref, d_ref):
    h = h_ref[...]
    s_ref[...] = jnp.dot(h, gs_ref[...], preferred_element_type=_f32)
    d_ref[...] = jnp.dot(h, gd_ref[...], preferred_element_type=_f32)


def _make_tables(hp, gs, gd):
    bn = 2048
    return pl.pallas_call(
        _tab_body,
        grid=(_NP // bn,),
        in_specs=[
            pl.BlockSpec((bn, _W), lambda i: (i, 0)),
            pl.BlockSpec((_W, _W), lambda i: (0, 0)),
            pl.BlockSpec((_W, _W), lambda i: (0, 0)),
        ],
        out_specs=[
            pl.BlockSpec((bn, _W), lambda i: (i, 0)),
            pl.BlockSpec((bn, _W), lambda i: (i, 0)),
        ],
        out_shape=[
            jax.ShapeDtypeStruct((_NP, _W), _f32),
            jax.ShapeDtypeStruct((_NP, _W), _f32),
        ],
    )(hp, gs, gd)


@functools.partial(
    pl.kernel,
    mesh=_mesh,
    out_type=[
        jax.ShapeDtypeStruct((_E, _W), _f32),
        jax.ShapeDtypeStruct((_E, _W), _f32),
        jax.ShapeDtypeStruct((_E, _W), _f32),
    ],
    scratch_types=[
        pltpu.VMEM((_N,), _f32),
        pltpu.VMEM((_N,), _f32),
        pltpu.VMEM((_N,), _f32),
        pltpu.VMEM((_CH,), jnp.int32),
        pltpu.VMEM((_CH,), jnp.int32),
        pltpu.VMEM((_CH, _W), _f32),
        pltpu.VMEM((_CH, _W), _f32),
        pltpu.VMEM((_CH, _W), _f32),
        pltpu.SemaphoreType.DMA,
        pltpu.SemaphoreType.DMA,
    ],
    compiler_params=pltpu.CompilerParams(needs_layout_passes=False),
)
def _gather_rows(stab, dtab, x0, x1, x2, sidx, didx,
                 orow_s, orow_d, oaux,
                 x0_t, x1_t, x2_t, iv_s, iv_d, rv_s, rv_d, bd,
                 sem_s, sem_d):
    pltpu.sync_copy(x0, x0_t)
    pltpu.sync_copy(x1, x1_t)
    pltpu.sync_copy(x2, x2_t)
    wid = lax.axis_index("s") * _NC + lax.axis_index("c")
    base = wid * _EW

    zv = jnp.zeros((16,), _f32)

    def zbody(r, carry):
        for j in range(_W // 16):
            bd[r, pl.ds(j * 16, 16)] = zv
        return carry

    lax.fori_loop(0, _CH, zbody, 0)

    def body(i, carry):
        cb = base + i * _CH
        pltpu.sync_copy(sidx.at[pl.ds(cb, _CH)], iv_s)
        pltpu.sync_copy(didx.at[pl.ds(cb, _CH)], iv_d)
        cp_s = pltpu.async_copy(stab.at[iv_s], rv_s, sem_s)
        cp_d = pltpu.async_copy(dtab.at[iv_d], rv_d, sem_d)
        for g in range(_CH // 16):
            sl = pl.ds(g * 16, 16)
            isv = iv_s[sl]
            idv = iv_d[sl]
            rows = g * 16 + jnp.arange(16, dtype=jnp.int32)
            for comp, xt in ((0, x0_t), (1, x1_t), (2, x2_t)):
                dv = plsc.load_gather(xt, [isv]) - plsc.load_gather(xt, [idv])
                hi = plsc.bitcast(
                    plsc.bitcast(dv, jnp.uint32) & jnp.uint32(0xFFFF0000), _f32)
                lo = dv - hi
                plsc.store_scatter(bd, [rows, jnp.full((16,), 2 * comp, jnp.int32)], hi)
                plsc.store_scatter(bd, [rows, jnp.full((16,), 2 * comp + 1, jnp.int32)], lo)
        cp_s.wait()
        cp_d.wait()
        pltpu.sync_copy(rv_s, orow_s.at[pl.ds(cb, _CH)])
        pltpu.sync_copy(rv_d, orow_d.at[pl.ds(cb, _CH)])
        pltpu.sync_copy(bd, oaux.at[pl.ds(cb, _CH)])
        return carry

    lax.fori_loop(0, _EW // _CH, body, 0)


def _edge_body(s_ref, d_ref, aux_ref, p0_ref, p1_ref, p2_ref, vsr_ref, vdr_ref,
               wf1_ref, bf1_ref, wf2_ref, bf2_ref, wew3_ref, bew_ref,
               bfin_ref, wc1_ref, bc1_ref, wc2r_ref, bc2_ref,
               o0_ref, o1_ref, o2_ref):
    s = s_ref[...]
    d = d_ref[...]
    aux = aux_ref[...]
    a = s[:, 0:64]
    cc = s[:, 64:128]
    b = d[:, 0:64]
    dd = d[:, 64:128]
    n = s.shape[0]

    dxb = jnp.dot(aux, p0_ref[...], preferred_element_type=_f32)
    dyb = jnp.dot(aux, p1_ref[...], preferred_element_type=_f32)
    dzb = jnp.dot(aux, p2_ref[...], preferred_element_type=_f32)
    d2 = dxb * dxb + dyb * dyb + dzb * dzb + _EPS
    r0 = lax.rsqrt(d2)
    inv = r0 * (1.5 - 0.5 * d2 * r0 * r0)
    dist = d2 * inv
    mu = (5.0 / 63.0) * lax.broadcasted_iota(jnp.int32, (1, 64), 1).astype(_f32)
    t = dist - mu
    rbf = jnp.exp(-10.0 * t * t)
    hf0 = (a + b + bfin_ref[...]) * rbf
    hf1 = jnp.dot(hf0, wf1_ref[...], preferred_element_type=_f32) + bf1_ref[...]
    hf = hf1 * jax.nn.sigmoid(hf1)
    he = jnp.dot(hf, wf2_ref[...], preferred_element_type=_f32) + bf2_ref[...]
    z = (jnp.dot(s, vsr_ref[...], preferred_element_type=_f32)
         + jnp.dot(d, vdr_ref[...], preferred_element_type=_f32))
    att = z * jax.nn.sigmoid(z)
    w = jnp.exp(att)
    ew = jnp.tanh(cc + dd + jnp.dot(he, wew3_ref[...], preferred_element_type=_f32) + bew_ref[...])
    c1 = jnp.dot(he, wc1_ref[...], preferred_element_type=_f32) + bc1_ref[...]
    c1 = c1 * jax.nn.sigmoid(c1)
    cw = jnp.dot(c1, wc2r_ref[...], preferred_element_type=_f32) + bc2_ref[...]
    o0_ref[...] = jnp.concatenate([w * he, ew * (dxb * inv)], axis=1)
    o1_ref[...] = jnp.concatenate([ew * (dyb * inv), ew * (dzb * inv)], axis=1)
    o2_ref[...] = jnp.concatenate(
        [w[:, 0:1], jnp.ones((n, 1), _f32),
         (cw * dxb)[:, 0:1], (cw * dyb)[:, 0:1], (cw * dzb)[:, 0:1],
         jnp.zeros((n, 123), _f32)], axis=1)


def _edge_compute(srow, drow, aux, p0, p1, p2, vsr, vdr,
                  wf1, bf1, wf2, bf2, wew3, bew, bfin, wc1, bc1, wc2r, bc2):
    be = 1600
    full = lambda r, c: pl.BlockSpec((r, c), lambda i: (0, 0))
    return pl.pallas_call(
        _edge_body,
        grid=(_E // be,),
        in_specs=[
            pl.BlockSpec((be, _W), lambda i: (i, 0)),
            pl.BlockSpec((be, _W), lambda i: (i, 0)),
            pl.BlockSpec((be, _W), lambda i: (i, 0)),
            full(_W, 64), full(_W, 64), full(_W, 64),
            full(_W, 64), full(_W, 64),
            full(64, 64), full(1, 64), full(64, 64), full(1, 64),
            full(64, 64), full(1, 64), full(1, 64),
            full(64, 64), full(1, 64), full(64, 64), full(1, 1),
        ],
        out_specs=[
            pl.BlockSpec((be, _W), lambda i: (i, 0)),
            pl.BlockSpec((be, _W), lambda i: (i, 0)),
            pl.BlockSpec((be, _W), lambda i: (i, 0)),
        ],
        out_shape=[
            jax.ShapeDtypeStruct((_E, _W), _f32),
            jax.ShapeDtypeStruct((_E, _W), _f32),
            jax.ShapeDtypeStruct((_E, _W), _f32),
        ],
    )(srow, drow, aux, p0, p1, p2, vsr, vdr,
      wf1, bf1, wf2, bf2, wew3, bew, bfin, wc1, bc1, wc2r, bc2)


@functools.partial(
    pl.kernel,
    mesh=_mesh,
    out_type=[
        jax.ShapeDtypeStruct((_NP, _W), _f32),
        jax.ShapeDtypeStruct((_NP, _W), _f32),
        jax.ShapeDtypeStruct((_NP, _W), _f32),
        jax.ShapeDtypeStruct((_NP, _W), _f32),
    ],
    scratch_types=[
        pltpu.VMEM((_CH,), jnp.int32),
        pltpu.VMEM((_CH, _W), _f32),
        pltpu.VMEM_SHARED((_NP, _W), _f32),
    ],
)
def _scatter_rows(o0, o1, o2, didx, zrow, acc0, acc1, acc2a, acc2b,
                  iv, rv, acc_sp):
    c = lax.axis_index("c")
    s = lax.axis_index("s")
    rb = s * _NROW

    def accumulate(edge_ref, ebase, nchunk):
        def body(i, carry):
            cb = ebase + i * _CH
            pltpu.sync_copy(didx.at[pl.ds(cb, _CH)], iv)
            pltpu.sync_copy(edge_ref.at[pl.ds(cb, _CH)], rv)
            pltpu.sync_copy(rv, acc_sp.at[iv], add=True)
            return carry
        lax.fori_loop(0, nchunk, body, 0)

    def flush(out_ref):
        pltpu.sync_copy(acc_sp.at[pl.ds(rb, _NROW)], out_ref.at[pl.ds(rb, _NROW)])

    pltpu.sync_copy(zrow, acc_sp.at[pl.ds(rb, _NROW)])
    plsc.subcore_barrier()

    @pl.when(c == 0)
    def _():
        accumulate(o0, s * _EC, _EC // _CH)

    @pl.when(c == 1)
    def _():
        accumulate(o1, s * _EC, _EC // _CH)

    plsc.subcore_barrier()

    @pl.when(c == 0)
    def _():
        flush(acc0)

    @pl.when(c == 1)
    def _():
        flush(acc1)

    plsc.subcore_barrier()

    pltpu.sync_copy(zrow, acc_sp.at[pl.ds(rb, _NROW)])
    plsc.subcore_barrier()
    accumulate(o2, c * (_E // 2) + s * _EH, _EH // _CH)
    plsc.subcore_barrier()

    @pl.when(c == 0)
    def _():
        flush(acc2a)

    @pl.when(c == 1)
    def _():
        flush(acc2b)


def _node_body(h_ref, x_ref, a0_ref, a1_ref, a2a_ref, a2b_ref,
               wpn1_ref, bpn1_ref, wpn2_ref, bpn2_ref,
               wn1a_ref, wn1b_ref, wn1c_ref, bn1_ref, wn2_ref, bn2_ref,
               hn_ref, xn_ref):
    a0 = a0_ref[...]
    a1 = a1_ref[...]
    a2 = a2a_ref[...] + a2b_ref[...]
    wsum = a2[:, 0:1]
    deg = a2[:, 1:2]
    cwd = a2[:, 2:5]
    heagg = a0[:, 0:64] / (wsum + _EPS)
    cx = a0[:, 64:128]
    cy = a1[:, 0:64]
    cz = a1[:, 64:128]
    cn = cx * cx + cy * cy + cz * cz
    t = jnp.dot(cn, wpn1_ref[...], preferred_element_type=_f32) + bpn1_ref[...]
    t = t * jax.nn.sigmoid(t)
    hcomb = jnp.dot(t, wpn2_ref[...], preferred_element_type=_f32) + bpn2_ref[...]
    h = h_ref[...]
    pre = (jnp.dot(h, wn1a_ref[...], preferred_element_type=_f32)
           + jnp.dot(heagg, wn1b_ref[...], preferred_element_type=_f32)
           + jnp.dot(hcomb, wn1c_ref[...], preferred_element_type=_f32)
           + bn1_ref[...])
    pre = pre * jax.nn.sigmoid(pre)
    hn_ref[...] = jnp.dot(pre, wn2_ref[...], preferred_element_type=_f32) + bn2_ref[...]
    xn_ref[...] = x_ref[...][:, 0:3] + cwd / (deg + 1.0)


def _node_out(h, xpad, acc0, acc1, acc2a, acc2b,
              wpn1, bpn1, wpn2, bpn2, wn1a, wn1b, wn1c, bn1, wn2, bn2):
    bn = 2000
    full = lambda r, c: pl.BlockSpec((r, c), lambda i: (0, 0))
    return pl.pallas_call(
        _node_body,
        grid=(_N // bn,),
        in_specs=[
            pl.BlockSpec((bn, _IN_F), lambda i: (i, 0)),
            pl.BlockSpec((bn, 8), lambda i: (i, 0)),
            pl.BlockSpec((bn, _W), lambda i: (i, 0)),
            pl.BlockSpec((bn, _W), lambda i: (i, 0)),
            pl.BlockSpec((bn, _W), lambda i: (i, 0)),
            pl.BlockSpec((bn, _W), lambda i: (i, 0)),
            full(64, 64), full(1, 64), full(64, 64), full(1, 64),
            full(128, 64), full(64, 64), full(64, 64), full(1, 64),
            full(64, 64), full(1, 64),
        ],
        out_specs=[
            pl.BlockSpec((bn, 64), lambda i: (i, 0)),
            pl.BlockSpec((bn, 3), lambda i: (i, 0)),
        ],
        out_shape=[
            jax.ShapeDtypeStruct((_N, 64), _f32),
            jax.ShapeDtypeStruct((_N, 3), _f32),
        ],
    )(h, xpad, acc0, acc1, acc2a, acc2b,
      wpn1, bpn1, wpn2, bpn2, wn1a, wn1b, wn1c, bn1, wn2, bn2)


def kernel(h, x, edge_index, W_fin, b_fin, W_f1, b_f1, W_f2, b_f2, W_sa,
           W_ew, b_ew, W_pn1, b_pn1, W_pn2, b_pn2, W_n1, b_n1, W_n2, b_n2,
           W_c1, b_c1, W_c2, b_c2):
    gs = jnp.concatenate([W_fin[0:128], W_ew[0:128]], axis=1)
    gd = jnp.concatenate([W_fin[128:256], W_ew[128:256]], axis=1)
    vs = jnp.linalg.solve(gs, W_sa[0:128])
    vd = jnp.linalg.solve(gd, W_sa[128:256])

    hp = jnp.pad(h, ((0, _NP - _N), (0, 0)))
    stab, dtab = _make_tables(hp, gs, gd)

    src = edge_index[0]
    dst = edge_index[1]
    x0 = x[:, 0]
    x1 = x[:, 1]
    x2 = x[:, 2]
    srow, drow, aux = _gather_rows(stab, dtab, x0, x1, x2, src, dst)

    ones64 = jnp.ones((1, 64), _f32)
    sel = (jnp.zeros((_W, 3), _f32)
           .at[0, 0].set(1.0).at[1, 0].set(1.0)
           .at[2, 1].set(1.0).at[3, 1].set(1.0)
           .at[4, 2].set(1.0).at[5, 2].set(1.0))
    p0 = sel[:, 0:1] * ones64
    p1 = sel[:, 1:2] * ones64
    p2 = sel[:, 2:3] * ones64
    o0, o1, o2 = _edge_compute(
        srow, drow, aux, p0, p1, p2, vs * ones64, vd * ones64,
        W_f1, b_f1[None, :], W_f2, b_f2[None, :],
        W_ew[256:320], b_ew[None, :], b_fin[None, :],
        W_c1, b_c1[None, :], W_c2 * ones64, b_c2[None, :])

    zrow = jnp.zeros((_NROW, _W), _f32)
    acc0, acc1, acc2a, acc2b = _scatter_rows(o0, o1, o2, dst, zrow)

    xpad = jnp.pad(x, ((0, 0), (0, 5)))
    h_new, x_new = _node_out(
        h, xpad, acc0, acc1, acc2a, acc2b,
        W_pn1, b_pn1[None, :], W_pn2, b_pn2[None, :],
        W_n1[0:128], W_n1[128:192], W_n1[192:256], b_n1[None, :],
        W_n2, b_n2[None, :])
    return h_new, x_new


# speedup vs baseline: 1.1850x; 1.1850x over previous
import functools

import jax
import jax.numpy as jnp
from jax import lax
from jax.experimental import pallas as pl
from jax.experimental.pallas import tpu as pltpu
from jax.experimental.pallas import tpu_sc as plsc

_N = 10000
_E = 320000
_IN_F = 128
_EPS = 1e-5
_NP = 10240
_W = 128

_f32 = jnp.float32

_info = plsc.get_sparse_core_info()
_NC = _info.num_cores
_NS = _info.num_subcores
_NW = _NC * _NS
_CH = 80
_EW = _E // _NW
_EC = _E // _NS
_EH = _E // 2 // _NS
_NROW = _NP // _NS

_mesh = plsc.VectorSubcoreMesh(core_axis_name="c", subcore_axis_name="s")


def _tab_body(h_ref, gs_ref, gd_ref, s_ref, d_ref):
    h = h_ref[...]
    s_ref[...] = jnp.dot(h, gs_ref[...], preferred_element_type=_f32)
    d_ref[...] = jnp.dot(h, gd_ref[...], preferred_element_type=_f32)


def _make_tables(hp, gs, gd):
    bn = 2048
    return pl.pallas_call(
        _tab_body,
        grid=(_NP // bn,),
        in_specs=[
            pl.BlockSpec((bn, _W), lambda i: (i, 0)),
            pl.BlockSpec((_W, _W), lambda i: (0, 0)),
            pl.BlockSpec((_W, _W), lambda i: (0, 0)),
        ],
        out_specs=[
            pl.BlockSpec((bn, _W), lambda i: (i, 0)),
            pl.BlockSpec((bn, _W), lambda i: (i, 0)),
        ],
        out_shape=[
            jax.ShapeDtypeStruct((_NP, _W), _f32),
            jax.ShapeDtypeStruct((_NP, _W), _f32),
        ],
    )(hp, gs, gd)


@functools.partial(
    pl.kernel,
    mesh=_mesh,
    out_type=[
        jax.ShapeDtypeStruct((_E, _W), _f32),
        jax.ShapeDtypeStruct((_E, _W), _f32),
        jax.ShapeDtypeStruct((_E, _W), _f32),
    ],
    scratch_types=[
        pltpu.VMEM((_N,), _f32),
        pltpu.VMEM((_N,), _f32),
        pltpu.VMEM((_N,), _f32),
        pltpu.VMEM((_CH,), jnp.int32),
        pltpu.VMEM((_CH,), jnp.int32),
        pltpu.VMEM((_CH, _W), _f32),
        pltpu.VMEM((_CH, _W), _f32),
        pltpu.VMEM((_CH, _W), _f32),
        pltpu.SemaphoreType.DMA,
        pltpu.SemaphoreType.DMA,
    ],
    compiler_params=pltpu.CompilerParams(needs_layout_passes=False),
)
def _gather_rows(stab, dtab, x0, x1, x2, sidx, didx,
                 orow_s, orow_d, oaux,
                 x0_t, x1_t, x2_t, iv_s, iv_d, rv_s, rv_d, bd,
                 sem_s, sem_d):
    pltpu.sync_copy(x0, x0_t)
    pltpu.sync_copy(x1, x1_t)
    pltpu.sync_copy(x2, x2_t)
    wid = lax.axis_index("s") * _NC + lax.axis_index("c")
    base = wid * _EW

    zv = jnp.zeros((16,), _f32)

    def zbody(r, carry):
        for j in range(_W // 16):
            bd[r, pl.ds(j * 16, 16)] = zv
        return carry

    lax.fori_loop(0, _CH, zbody, 0)

    def body(i, carry):
        cb = base + i * _CH
        pltpu.sync_copy(sidx.at[pl.ds(cb, _CH)], iv_s)
        pltpu.sync_copy(didx.at[pl.ds(cb, _CH)], iv_d)
        cp_s = pltpu.async_copy(stab.at[iv_s], rv_s, sem_s)
        cp_d = pltpu.async_copy(dtab.at[iv_d], rv_d, sem_d)
        for g in range(_CH // 16):
            sl = pl.ds(g * 16, 16)
            isv = iv_s[sl]
            idv = iv_d[sl]
            rows = g * 16 + jnp.arange(16, dtype=jnp.int32)
            for comp, xt in ((0, x0_t), (1, x1_t), (2, x2_t)):
                dv = plsc.load_gather(xt, [isv]) - plsc.load_gather(xt, [idv])
                hi = plsc.bitcast(
                    plsc.bitcast(dv, jnp.uint32) & jnp.uint32(0xFFFF0000), _f32)
                lo = dv - hi
                plsc.store_scatter(bd, [rows, jnp.full((16,), 2 * comp, jnp.int32)], hi)
                plsc.store_scatter(bd, [rows, jnp.full((16,), 2 * comp + 1, jnp.int32)], lo)
        cp_s.wait()
        cp_d.wait()
        pltpu.sync_copy(rv_s, orow_s.at[pl.ds(cb, _CH)])
        pltpu.sync_copy(rv_d, orow_d.at[pl.ds(cb, _CH)])
        pltpu.sync_copy(bd, oaux.at[pl.ds(cb, _CH)])
        return carry

    lax.fori_loop(0, _EW // _CH, body, 0)


def _edge_body(s_ref, d_ref, aux_ref, p0_ref, p1_ref, p2_ref, vsr_ref, vdr_ref,
               wf1_ref, bf1_ref, wf2_ref, bf2_ref, wew3_ref, bew_ref,
               bfin_ref, wc1_ref, bc1_ref, wc2r_ref, bc2_ref,
               o0_ref, o1_ref, o2_ref):
    s = s_ref[...]
    d = d_ref[...]
    aux = aux_ref[...]
    a = s[:, 0:64]
    cc = s[:, 64:128]
    b = d[:, 0:64]
    dd = d[:, 64:128]
    n = s.shape[0]

    dxb = jnp.dot(aux, p0_ref[...], preferred_element_type=_f32)
    dyb = jnp.dot(aux, p1_ref[...], preferred_element_type=_f32)
    dzb = jnp.dot(aux, p2_ref[...], preferred_element_type=_f32)
    d2 = dxb * dxb + dyb * dyb + dzb * dzb + _EPS
    r0 = lax.rsqrt(d2)
    inv = r0 * (1.5 - 0.5 * d2 * r0 * r0)
    dist = d2 * inv
    mu = (5.0 / 63.0) * lax.broadcasted_iota(jnp.int32, (1, 64), 1).astype(_f32)
    t = dist - mu
    rbf = jnp.exp(-10.0 * t * t)
    hf0 = (a + b + bfin_ref[...]) * rbf
    hf1 = jnp.dot(hf0, wf1_ref[...], preferred_element_type=_f32) + bf1_ref[...]
    hf = hf1 * jax.nn.sigmoid(hf1)
    he = jnp.dot(hf, wf2_ref[...], preferred_element_type=_f32) + bf2_ref[...]
    z = (jnp.dot(s, vsr_ref[...], preferred_element_type=_f32)
         + jnp.dot(d, vdr_ref[...], preferred_element_type=_f32))
    att = z * jax.nn.sigmoid(z)
    w = jnp.exp(att)
    ew = jnp.tanh(cc + dd + jnp.dot(he, wew3_ref[...], preferred_element_type=_f32) + bew_ref[...])
    c1 = jnp.dot(he, wc1_ref[...], preferred_element_type=_f32) + bc1_ref[...]
    c1 = c1 * jax.nn.sigmoid(c1)
    cw = jnp.dot(c1, wc2r_ref[...], preferred_element_type=_f32) + bc2_ref[...]
    o0_ref[...] = jnp.concatenate([w * he, ew * (dxb * inv)], axis=1)
    o1_ref[...] = jnp.concatenate([ew * (dyb * inv), ew * (dzb * inv)], axis=1)
    o2_ref[...] = jnp.concatenate(
        [w[:, 0:1], jnp.ones((n, 1), _f32),
         (cw * dxb)[:, 0:1], (cw * dyb)[:, 0:1], (cw * dzb)[:, 0:1],
         jnp.zeros((n, 123), _f32)], axis=1)


def _edge_compute(srow, drow, aux, p0, p1, p2, vsr, vdr,
                  wf1, bf1, wf2, bf2, wew3, bew, bfin, wc1, bc1, wc2r, bc2):
    be = 1600
    full = lambda r, c: pl.BlockSpec((r, c), lambda i: (0, 0))
    return pl.pallas_call(
        _edge_body,
        grid=(_E // be,),
        in_specs=[
            pl.BlockSpec((be, _W), lambda i: (i, 0)),
            pl.BlockSpec((be, _W), lambda i: (i, 0)),
            pl.BlockSpec((be, _W), lambda i: (i, 0)),
            full(_W, 64), full(_W, 64), full(_W, 64),
            full(_W, 64), full(_W, 64),
            full(64, 64), full(1, 64), full(64, 64), full(1, 64),
            full(64, 64), full(1, 64), full(1, 64),
            full(64, 64), full(1, 64), full(64, 64), full(1, 1),
        ],
        out_specs=[
            pl.BlockSpec((be, _W), lambda i: (i, 0)),
            pl.BlockSpec((be, _W), lambda i: (i, 0)),
            pl.BlockSpec((be, _W), lambda i: (i, 0)),
        ],
        out_shape=[
            jax.ShapeDtypeStruct((_E, _W), _f32),
            jax.ShapeDtypeStruct((_E, _W), _f32),
            jax.ShapeDtypeStruct((_E, _W), _f32),
        ],
    )(srow, drow, aux, p0, p1, p2, vsr, vdr,
      wf1, bf1, wf2, bf2, wew3, bew, bfin, wc1, bc1, wc2r, bc2)


@functools.partial(
    pl.kernel,
    mesh=_mesh,
    out_type=[
        jax.ShapeDtypeStruct((_NP, _W), _f32),
        jax.ShapeDtypeStruct((_NP, _W), _f32),
        jax.ShapeDtypeStruct((_NP, _W), _f32),
        jax.ShapeDtypeStruct((_NP, _W), _f32),
    ],
    scratch_types=[
        pltpu.VMEM((_CH,), jnp.int32),
        pltpu.VMEM((_CH,), jnp.int32),
        pltpu.VMEM((_CH, _W), _f32),
        pltpu.VMEM((_CH, _W), _f32),
        pltpu.VMEM_SHARED((_NP, _W), _f32),
        pltpu.SemaphoreType.DMA,
        pltpu.SemaphoreType.DMA,
        pltpu.SemaphoreType.DMA,
        pltpu.SemaphoreType.DMA,
    ],
)
def _scatter_rows(o0, o1, o2, didx, zrow, acc0, acc1, acc2a, acc2b,
                  iv_a, iv_b, rv_a, rv_b, acc_sp, sem_a, sem_b, sem_sa, sem_sb):
    c = lax.axis_index("c")
    s = lax.axis_index("s")
    rb = s * _NROW

    def accumulate(edge_ref, ebase, nchunk):
        def cbase(k):
            return ebase + jnp.minimum(k, nchunk - 1) * _CH

        def pf(k, iv, rv, sem):
            cb = cbase(k)
            pltpu.async_copy(didx.at[pl.ds(cb, _CH)], iv, sem)
            pltpu.async_copy(edge_ref.at[pl.ds(cb, _CH)], rv, sem)

        def drain_pf(k, iv, rv, sem):
            cb = cbase(k)
            pltpu.make_async_copy(didx.at[pl.ds(cb, _CH)], iv, sem).wait()
            pltpu.make_async_copy(edge_ref.at[pl.ds(cb, _CH)], rv, sem).wait()

        npairs = (nchunk + 1) // 2
        pf(0, iv_a, rv_a, sem_a)
        pf(1, iv_b, rv_b, sem_b)

        def body(j, carry):
            c0 = 2 * j
            c1 = c0 + 1
            drain_pf(c0, iv_a, rv_a, sem_a)
            sca = pltpu.async_copy(rv_a, acc_sp.at[iv_a], sem_sa, add=True)
            drain_pf(c1, iv_b, rv_b, sem_b)

            @pl.when(c1 < nchunk)
            def _():
                pltpu.async_copy(rv_b, acc_sp.at[iv_b], sem_sb, add=True)

            sca.wait()
            pf(c0 + 2, iv_a, rv_a, sem_a)

            @pl.when(c1 < nchunk)
            def _():
                pltpu.make_async_copy(rv_b, acc_sp.at[iv_b], sem_sb).wait()

            pf(c1 + 2, iv_b, rv_b, sem_b)
            return carry

        lax.fori_loop(0, npairs, body, 0)
        drain_pf(2 * npairs, iv_a, rv_a, sem_a)
        drain_pf(2 * npairs + 1, iv_b, rv_b, sem_b)

    def flush(out_ref):
        pltpu.sync_copy(acc_sp.at[pl.ds(rb, _NROW)], out_ref.at[pl.ds(rb, _NROW)])

    pltpu.sync_copy(zrow, acc_sp.at[pl.ds(rb, _NROW)])
    plsc.subcore_barrier()

    @pl.when(c == 0)
    def _():
        accumulate(o0, s * _EC, _EC // _CH)

    @pl.when(c == 1)
    def _():
        accumulate(o1, s * _EC, _EC // _CH)

    plsc.subcore_barrier()

    @pl.when(c == 0)
    def _():
        flush(acc0)

    @pl.when(c == 1)
    def _():
        flush(acc1)

    plsc.subcore_barrier()

    pltpu.sync_copy(zrow, acc_sp.at[pl.ds(rb, _NROW)])
    plsc.subcore_barrier()
    accumulate(o2, c * (_E // 2) + s * _EH, _EH // _CH)
    plsc.subcore_barrier()

    @pl.when(c == 0)
    def _():
        flush(acc2a)

    @pl.when(c == 1)
    def _():
        flush(acc2b)


def _node_body(h_ref, x_ref, a0_ref, a1_ref, a2a_ref, a2b_ref,
               wpn1_ref, bpn1_ref, wpn2_ref, bpn2_ref,
               wn1a_ref, wn1b_ref, wn1c_ref, bn1_ref, wn2_ref, bn2_ref,
               hn_ref, xn_ref):
    a0 = a0_ref[...]
    a1 = a1_ref[...]
    a2 = a2a_ref[...] + a2b_ref[...]
    wsum = a2[:, 0:1]
    deg = a2[:, 1:2]
    cwd = a2[:, 2:5]
    heagg = a0[:, 0:64] / (wsum + _EPS)
    cx = a0[:, 64:128]
    cy = a1[:, 0:64]
    cz = a1[:, 64:128]
    cn = cx * cx + cy * cy + cz * cz
    t = jnp.dot(cn, wpn1_ref[...], preferred_element_type=_f32) + bpn1_ref[...]
    t = t * jax.nn.sigmoid(t)
    hcomb = jnp.dot(t, wpn2_ref[...], preferred_element_type=_f32) + bpn2_ref[...]
    h = h_ref[...]
    pre = (jnp.dot(h, wn1a_ref[...], preferred_element_type=_f32)
           + jnp.dot(heagg, wn1b_ref[...], preferred_element_type=_f32)
           + jnp.dot(hcomb, wn1c_ref[...], preferred_element_type=_f32)
           + bn1_ref[...])
    pre = pre * jax.nn.sigmoid(pre)
    hn_ref[...] = jnp.dot(pre, wn2_ref[...], preferred_element_type=_f32) + bn2_ref[...]
    xn_ref[...] = x_ref[...][:, 0:3] + cwd / (deg + 1.0)


def _node_out(h, xpad, acc0, acc1, acc2a, acc2b,
              wpn1, bpn1, wpn2, bpn2, wn1a, wn1b, wn1c, bn1, wn2, bn2):
    bn = 2000
    full = lambda r, c: pl.BlockSpec((r, c), lambda i: (0, 0))
    return pl.pallas_call(
        _node_body,
        grid=(_N // bn,),
        in_specs=[
            pl.BlockSpec((bn, _IN_F), lambda i: (i, 0)),
            pl.BlockSpec((bn, 8), lambda i: (i, 0)),
            pl.BlockSpec((bn, _W), lambda i: (i, 0)),
            pl.BlockSpec((bn, _W), lambda i: (i, 0)),
            pl.BlockSpec((bn, _W), lambda i: (i, 0)),
            pl.BlockSpec((bn, _W), lambda i: (i, 0)),
            full(64, 64), full(1, 64), full(64, 64), full(1, 64),
            full(128, 64), full(64, 64), full(64, 64), full(1, 64),
            full(64, 64), full(1, 64),
        ],
        out_specs=[
            pl.BlockSpec((bn, 64), lambda i: (i, 0)),
            pl.BlockSpec((bn, 3), lambda i: (i, 0)),
        ],
        out_shape=[
            jax.ShapeDtypeStruct((_N, 64), _f32),
            jax.ShapeDtypeStruct((_N, 3), _f32),
        ],
    )(h, xpad, acc0, acc1, acc2a, acc2b,
      wpn1, bpn1, wpn2, bpn2, wn1a, wn1b, wn1c, bn1, wn2, bn2)


def kernel(h, x, edge_index, W_fin, b_fin, W_f1, b_f1, W_f2, b_f2, W_sa,
           W_ew, b_ew, W_pn1, b_pn1, W_pn2, b_pn2, W_n1, b_n1, W_n2, b_n2,
           W_c1, b_c1, W_c2, b_c2):
    gs = jnp.concatenate([W_fin[0:128], W_ew[0:128]], axis=1)
    gd = jnp.concatenate([W_fin[128:256], W_ew[128:256]], axis=1)
    vs = jnp.linalg.solve(gs, W_sa[0:128])
    vd = jnp.linalg.solve(gd, W_sa[128:256])

    hp = jnp.pad(h, ((0, _NP - _N), (0, 0)))
    stab, dtab = _make_tables(hp, gs, gd)

    src = edge_index[0]
    dst = edge_index[1]
    x0 = x[:, 0]
    x1 = x[:, 1]
    x2 = x[:, 2]
    srow, drow, aux = _gather_rows(stab, dtab, x0, x1, x2, src, dst)

    ones64 = jnp.ones((1, 64), _f32)
    sel = (jnp.zeros((_W, 3), _f32)
           .at[0, 0].set(1.0).at[1, 0].set(1.0)
           .at[2, 1].set(1.0).at[3, 1].set(1.0)
           .at[4, 2].set(1.0).at[5, 2].set(1.0))
    p0 = sel[:, 0:1] * ones64
    p1 = sel[:, 1:2] * ones64
    p2 = sel[:, 2:3] * ones64
    o0, o1, o2 = _edge_compute(
        srow, drow, aux, p0, p1, p2, vs * ones64, vd * ones64,
        W_f1, b_f1[None, :], W_f2, b_f2[None, :],
        W_ew[256:320], b_ew[None, :], b_fin[None, :],
        W_c1, b_c1[None, :], W_c2 * ones64, b_c2[None, :])

    zrow = jnp.zeros((_NROW, _W), _f32)
    acc0, acc1, acc2a, acc2b = _scatter_rows(o0, o1, o2, dst, zrow)

    xpad = jnp.pad(x, ((0, 0), (0, 5)))
    h_new, x_new = _node_out(
        h, xpad, acc0, acc1, acc2a, acc2b,
        W_pn1, b_pn1[None, :], W_pn2, b_pn2[None, :],
        W_n1[0:128], W_n1[128:192], W_n1[192:256], b_n1[None, :],
        W_n2, b_n2[None, :])
    return h_new, x_new


# speedup vs baseline: 1.3273x; 1.1201x over previous
import functools

import jax
import jax.numpy as jnp
from jax import lax
from jax.experimental import pallas as pl
from jax.experimental.pallas import tpu as pltpu
from jax.experimental.pallas import tpu_sc as plsc

_N = 10000
_E = 320000
_IN_F = 128
_EPS = 1e-5
_NP = 10240
_W = 128

_f32 = jnp.float32

_info = plsc.get_sparse_core_info()
_NC = _info.num_cores
_NS = _info.num_subcores
_NW = _NC * _NS
_CH = 80
_EW = _E // _NW
_EC = _E // _NS
_EH = _E // 2 // _NS
_NROW = _NP // _NS

_mesh = plsc.VectorSubcoreMesh(core_axis_name="c", subcore_axis_name="s")


def _tab_body(h_ref, gs_ref, gd_ref, s_ref, d_ref):
    h = h_ref[...]
    s_ref[...] = jnp.dot(h, gs_ref[...], preferred_element_type=_f32)
    d_ref[...] = jnp.dot(h, gd_ref[...], preferred_element_type=_f32)


def _make_tables(hp, gs, gd):
    bn = 2048
    return pl.pallas_call(
        _tab_body,
        grid=(_NP // bn,),
        in_specs=[
            pl.BlockSpec((bn, _W), lambda i: (i, 0)),
            pl.BlockSpec((_W, _W), lambda i: (0, 0)),
            pl.BlockSpec((_W, _W), lambda i: (0, 0)),
        ],
        out_specs=[
            pl.BlockSpec((bn, _W), lambda i: (i, 0)),
            pl.BlockSpec((bn, _W), lambda i: (i, 0)),
        ],
        out_shape=[
            jax.ShapeDtypeStruct((_NP, _W), _f32),
            jax.ShapeDtypeStruct((_NP, _W), _f32),
        ],
    )(hp, gs, gd)


@functools.partial(
    pl.kernel,
    mesh=_mesh,
    out_type=[
        jax.ShapeDtypeStruct((_E, _W), _f32),
        jax.ShapeDtypeStruct((_E, _W), _f32),
        jax.ShapeDtypeStruct((_E, _W), _f32),
    ],
    scratch_types=[
        pltpu.VMEM((_N,), _f32),
        pltpu.VMEM((_N,), _f32),
        pltpu.VMEM((_N,), _f32),
        pltpu.VMEM((2, _CH), jnp.int32),
        pltpu.VMEM((2, _CH), jnp.int32),
        pltpu.VMEM((2, _CH, _W), _f32),
        pltpu.VMEM((2, _CH, _W), _f32),
        pltpu.VMEM((2, _CH, _W), _f32),
        pltpu.SemaphoreType.DMA,
        pltpu.SemaphoreType.DMA,
        pltpu.SemaphoreType.DMA,
        pltpu.SemaphoreType.DMA,
        pltpu.SemaphoreType.DMA,
        pltpu.SemaphoreType.DMA,
    ],
    compiler_params=pltpu.CompilerParams(needs_layout_passes=False),
)
def _gather_rows(stab, dtab, x0, x1, x2, sidx, didx,
                 orow_s, orow_d, oaux,
                 x0_t, x1_t, x2_t, iv_s2, iv_d2, rv_s2, rv_d2, bd2,
                 sem_i0, sem_i1, sem_g0, sem_g1, sem_w0, sem_w1):
    pltpu.sync_copy(x0, x0_t)
    pltpu.sync_copy(x1, x1_t)
    pltpu.sync_copy(x2, x2_t)
    wid = lax.axis_index("s") * _NC + lax.axis_index("c")
    base = wid * _EW
    nchunk = _EW // _CH
    npairs = (nchunk + 1) // 2
    sem_i = (sem_i0, sem_i1)
    sem_g = (sem_g0, sem_g1)
    sem_w = (sem_w0, sem_w1)

    zv = jnp.zeros((16,), _f32)

    def zbody(r, carry):
        for p in range(2):
            for j in range(_W // 16):
                bd2[p, r, pl.ds(j * 16, 16)] = zv
        return carry

    lax.fori_loop(0, _CH, zbody, 0)

    def cbase(k):
        return base + jnp.minimum(k, nchunk - 1) * _CH

    def pf_idx(k, p):
        cb = cbase(k)
        pltpu.async_copy(sidx.at[pl.ds(cb, _CH)], iv_s2.at[p], sem_i[p])
        pltpu.async_copy(didx.at[pl.ds(cb, _CH)], iv_d2.at[p], sem_i[p])

    def drain_idx(k, p):
        cb = cbase(k)
        pltpu.make_async_copy(sidx.at[pl.ds(cb, _CH)], iv_s2.at[p], sem_i[p]).wait()
        pltpu.make_async_copy(didx.at[pl.ds(cb, _CH)], iv_d2.at[p], sem_i[p]).wait()

    def start_gathers(p):
        pltpu.async_copy(stab.at[iv_s2.at[p]], rv_s2.at[p], sem_g[p])
        pltpu.async_copy(dtab.at[iv_d2.at[p]], rv_d2.at[p], sem_g[p])

    def wait_gathers(p):
        pltpu.make_async_copy(stab.at[iv_s2.at[p]], rv_s2.at[p], sem_g[p]).wait()
        pltpu.make_async_copy(dtab.at[iv_d2.at[p]], rv_d2.at[p], sem_g[p]).wait()

    def aux_compute(p):
        for g in range(_CH // 16):
            sl = pl.ds(g * 16, 16)
            isv = iv_s2[p, sl]
            idv = iv_d2[p, sl]
            rows = g * 16 + jnp.arange(16, dtype=jnp.int32)
            for comp, xt in ((0, x0_t), (1, x1_t), (2, x2_t)):
                dv = plsc.load_gather(xt, [isv]) - plsc.load_gather(xt, [idv])
                hi = plsc.bitcast(
                    plsc.bitcast(dv, jnp.uint32) & jnp.uint32(0xFFFF0000), _f32)
                lo = dv - hi
                plsc.store_scatter(
                    bd2.at[p], [rows, jnp.full((16,), 2 * comp, jnp.int32)], hi)
                plsc.store_scatter(
                    bd2.at[p], [rows, jnp.full((16,), 2 * comp + 1, jnp.int32)], lo)

    def start_wb(k, p):
        cb = cbase(k)
        pltpu.async_copy(rv_s2.at[p], orow_s.at[pl.ds(cb, _CH)], sem_w[p])
        pltpu.async_copy(rv_d2.at[p], orow_d.at[pl.ds(cb, _CH)], sem_w[p])
        pltpu.async_copy(bd2.at[p], oaux.at[pl.ds(cb, _CH)], sem_w[p])

    def wait_wb(k, p):
        cb = cbase(k)
        pltpu.make_async_copy(rv_s2.at[p], orow_s.at[pl.ds(cb, _CH)], sem_w[p]).wait()
        pltpu.make_async_copy(rv_d2.at[p], orow_d.at[pl.ds(cb, _CH)], sem_w[p]).wait()
        pltpu.make_async_copy(bd2.at[p], oaux.at[pl.ds(cb, _CH)], sem_w[p]).wait()

    pf_idx(0, 0)
    pf_idx(1, 1)

    def body(j, carry):
        c0 = 2 * j
        c1 = c0 + 1
        drain_idx(c0, 0)
        start_gathers(0)
        aux_compute(0)
        drain_idx(c1, 1)
        start_gathers(1)
        aux_compute(1)
        wait_gathers(0)
        start_wb(c0, 0)
        pf_idx(c0 + 2, 0)
        wait_gathers(1)
        start_wb(c1, 1)
        pf_idx(c1 + 2, 1)
        wait_wb(c0, 0)
        wait_wb(c1, 1)
        return carry

    lax.fori_loop(0, npairs, body, 0)
    drain_idx(2 * npairs, 0)
    drain_idx(2 * npairs + 1, 1)


def _edge_body(s_ref, d_ref, aux_ref, p0_ref, p1_ref, p2_ref, vsr_ref, vdr_ref,
               wf1_ref, bf1_ref, wf2_ref, bf2_ref, wew3_ref, bew_ref,
               bfin_ref, wc1_ref, bc1_ref, wc2r_ref, bc2_ref,
               o0_ref, o1_ref, o2_ref):
    s = s_ref[...]
    d = d_ref[...]
    aux = aux_ref[...]
    a = s[:, 0:64]
    cc = s[:, 64:128]
    b = d[:, 0:64]
    dd = d[:, 64:128]
    n = s.shape[0]

    dxb = jnp.dot(aux, p0_ref[...], preferred_element_type=_f32)
    dyb = jnp.dot(aux, p1_ref[...], preferred_element_type=_f32)
    dzb = jnp.dot(aux, p2_ref[...], preferred_element_type=_f32)
    d2 = dxb * dxb + dyb * dyb + dzb * dzb + _EPS
    r0 = lax.rsqrt(d2)
    inv = r0 * (1.5 - 0.5 * d2 * r0 * r0)
    dist = d2 * inv
    mu = (5.0 / 63.0) * lax.broadcasted_iota(jnp.int32, (1, 64), 1).astype(_f32)
    t = dist - mu
    rbf = jnp.exp(-10.0 * t * t)
    hf0 = (a + b + bfin_ref[...]) * rbf
    hf1 = jnp.dot(hf0, wf1_ref[...], preferred_element_type=_f32) + bf1_ref[...]
    hf = hf1 * jax.nn.sigmoid(hf1)
    he = jnp.dot(hf, wf2_ref[...], preferred_element_type=_f32) + bf2_ref[...]
    z = (jnp.dot(s, vsr_ref[...], preferred_element_type=_f32)
         + jnp.dot(d, vdr_ref[...], preferred_element_type=_f32))
    att = z * jax.nn.sigmoid(z)
    w = jnp.exp(att)
    ew = jnp.tanh(cc + dd + jnp.dot(he, wew3_ref[...], preferred_element_type=_f32) + bew_ref[...])
    c1 = jnp.dot(he, wc1_ref[...], preferred_element_type=_f32) + bc1_ref[...]
    c1 = c1 * jax.nn.sigmoid(c1)
    cw = jnp.dot(c1, wc2r_ref[...], preferred_element_type=_f32) + bc2_ref[...]
    o0_ref[...] = jnp.concatenate([w * he, ew * (dxb * inv)], axis=1)
    o1_ref[...] = jnp.concatenate([ew * (dyb * inv), ew * (dzb * inv)], axis=1)
    o2_ref[...] = jnp.concatenate(
        [w[:, 0:1], jnp.ones((n, 1), _f32),
         (cw * dxb)[:, 0:1], (cw * dyb)[:, 0:1], (cw * dzb)[:, 0:1],
         jnp.zeros((n, 123), _f32)], axis=1)


def _edge_compute(srow, drow, aux, p0, p1, p2, vsr, vdr,
                  wf1, bf1, wf2, bf2, wew3, bew, bfin, wc1, bc1, wc2r, bc2):
    be = 1600
    full = lambda r, c: pl.BlockSpec((r, c), lambda i: (0, 0))
    return pl.pallas_call(
        _edge_body,
        grid=(_E // be,),
        in_specs=[
            pl.BlockSpec((be, _W), lambda i: (i, 0)),
            pl.BlockSpec((be, _W), lambda i: (i, 0)),
            pl.BlockSpec((be, _W), lambda i: (i, 0)),
            full(_W, 64), full(_W, 64), full(_W, 64),
            full(_W, 64), full(_W, 64),
            full(64, 64), full(1, 64), full(64, 64), full(1, 64),
            full(64, 64), full(1, 64), full(1, 64),
            full(64, 64), full(1, 64), full(64, 64), full(1, 1),
        ],
        out_specs=[
            pl.BlockSpec((be, _W), lambda i: (i, 0)),
            pl.BlockSpec((be, _W), lambda i: (i, 0)),
            pl.BlockSpec((be, _W), lambda i: (i, 0)),
        ],
        out_shape=[
            jax.ShapeDtypeStruct((_E, _W), _f32),
            jax.ShapeDtypeStruct((_E, _W), _f32),
            jax.ShapeDtypeStruct((_E, _W), _f32),
        ],
    )(srow, drow, aux, p0, p1, p2, vsr, vdr,
      wf1, bf1, wf2, bf2, wew3, bew, bfin, wc1, bc1, wc2r, bc2)


@functools.partial(
    pl.kernel,
    mesh=_mesh,
    out_type=[
        jax.ShapeDtypeStruct((_NP, _W), _f32),
        jax.ShapeDtypeStruct((_NP, _W), _f32),
        jax.ShapeDtypeStruct((_NP, _W), _f32),
        jax.ShapeDtypeStruct((_NP, _W), _f32),
    ],
    scratch_types=[
        pltpu.VMEM((_CH,), jnp.int32),
        pltpu.VMEM((_CH,), jnp.int32),
        pltpu.VMEM((_CH, _W), _f32),
        pltpu.VMEM((_CH, _W), _f32),
        pltpu.VMEM_SHARED((_NP, _W), _f32),
        pltpu.SemaphoreType.DMA,
        pltpu.SemaphoreType.DMA,
        pltpu.SemaphoreType.DMA,
        pltpu.SemaphoreType.DMA,
    ],
)
def _scatter_rows(o0, o1, o2, didx, zrow, acc0, acc1, acc2a, acc2b,
                  iv_a, iv_b, rv_a, rv_b, acc_sp, sem_a, sem_b, sem_sa, sem_sb):
    c = lax.axis_index("c")
    s = lax.axis_index("s")
    rb = s * _NROW

    def accumulate(edge_ref, ebase, nchunk):
        def cbase(k):
            return ebase + jnp.minimum(k, nchunk - 1) * _CH

        def pf(k, iv, rv, sem):
            cb = cbase(k)
            pltpu.async_copy(didx.at[pl.ds(cb, _CH)], iv, sem)
            pltpu.async_copy(edge_ref.at[pl.ds(cb, _CH)], rv, sem)

        def drain_pf(k, iv, rv, sem):
            cb = cbase(k)
            pltpu.make_async_copy(didx.at[pl.ds(cb, _CH)], iv, sem).wait()
            pltpu.make_async_copy(edge_ref.at[pl.ds(cb, _CH)], rv, sem).wait()

        npairs = (nchunk + 1) // 2
        pf(0, iv_a, rv_a, sem_a)
        pf(1, iv_b, rv_b, sem_b)

        def body(j, carry):
            c0 = 2 * j
            c1 = c0 + 1
            drain_pf(c0, iv_a, rv_a, sem_a)
            sca = pltpu.async_copy(rv_a, acc_sp.at[iv_a], sem_sa, add=True)
            drain_pf(c1, iv_b, rv_b, sem_b)

            @pl.when(c1 < nchunk)
            def _():
                pltpu.async_copy(rv_b, acc_sp.at[iv_b], sem_sb, add=True)

            sca.wait()
            pf(c0 + 2, iv_a, rv_a, sem_a)

            @pl.when(c1 < nchunk)
            def _():
                pltpu.make_async_copy(rv_b, acc_sp.at[iv_b], sem_sb).wait()

            pf(c1 + 2, iv_b, rv_b, sem_b)
            return carry

        lax.fori_loop(0, npairs, body, 0)
        drain_pf(2 * npairs, iv_a, rv_a, sem_a)
        drain_pf(2 * npairs + 1, iv_b, rv_b, sem_b)

    def flush(out_ref):
        pltpu.sync_copy(acc_sp.at[pl.ds(rb, _NROW)], out_ref.at[pl.ds(rb, _NROW)])

    pltpu.sync_copy(zrow, acc_sp.at[pl.ds(rb, _NROW)])
    plsc.subcore_barrier()

    @pl.when(c == 0)
    def _():
        accumulate(o0, s * _EC, _EC // _CH)

    @pl.when(c == 1)
    def _():
        accumulate(o1, s * _EC, _EC // _CH)

    plsc.subcore_barrier()

    @pl.when(c == 0)
    def _():
        flush(acc0)

    @pl.when(c == 1)
    def _():
        flush(acc1)

    plsc.subcore_barrier()

    pltpu.sync_copy(zrow, acc_sp.at[pl.ds(rb, _NROW)])
    plsc.subcore_barrier()
    accumulate(o2, c * (_E // 2) + s * _EH, _EH // _CH)
    plsc.subcore_barrier()

    @pl.when(c == 0)
    def _():
        flush(acc2a)

    @pl.when(c == 1)
    def _():
        flush(acc2b)


def _node_body(h_ref, x_ref, a0_ref, a1_ref, a2a_ref, a2b_ref,
               wpn1_ref, bpn1_ref, wpn2_ref, bpn2_ref,
               wn1a_ref, wn1b_ref, wn1c_ref, bn1_ref, wn2_ref, bn2_ref,
               hn_ref, xn_ref):
    a0 = a0_ref[...]
    a1 = a1_ref[...]
    a2 = a2a_ref[...] + a2b_ref[...]
    wsum = a2[:, 0:1]
    deg = a2[:, 1:2]
    cwd = a2[:, 2:5]
    heagg = a0[:, 0:64] / (wsum + _EPS)
    cx = a0[:, 64:128]
    cy = a1[:, 0:64]
    cz = a1[:, 64:128]
    cn = cx * cx + cy * cy + cz * cz
    t = jnp.dot(cn, wpn1_ref[...], preferred_element_type=_f32) + bpn1_ref[...]
    t = t * jax.nn.sigmoid(t)
    hcomb = jnp.dot(t, wpn2_ref[...], preferred_element_type=_f32) + bpn2_ref[...]
    h = h_ref[...]
    pre = (jnp.dot(h, wn1a_ref[...], preferred_element_type=_f32)
           + jnp.dot(heagg, wn1b_ref[...], preferred_element_type=_f32)
           + jnp.dot(hcomb, wn1c_ref[...], preferred_element_type=_f32)
           + bn1_ref[...])
    pre = pre * jax.nn.sigmoid(pre)
    hn_ref[...] = jnp.dot(pre, wn2_ref[...], preferred_element_type=_f32) + bn2_ref[...]
    xn_ref[...] = x_ref[...][:, 0:3] + cwd / (deg + 1.0)


def _node_out(h, xpad, acc0, acc1, acc2a, acc2b,
              wpn1, bpn1, wpn2, bpn2, wn1a, wn1b, wn1c, bn1, wn2, bn2):
    bn = 2000
    full = lambda r, c: pl.BlockSpec((r, c), lambda i: (0, 0))
    return pl.pallas_call(
        _node_body,
        grid=(_N // bn,),
        in_specs=[
            pl.BlockSpec((bn, _IN_F), lambda i: (i, 0)),
            pl.BlockSpec((bn, 8), lambda i: (i, 0)),
            pl.BlockSpec((bn, _W), lambda i: (i, 0)),
            pl.BlockSpec((bn, _W), lambda i: (i, 0)),
            pl.BlockSpec((bn, _W), lambda i: (i, 0)),
            pl.BlockSpec((bn, _W), lambda i: (i, 0)),
            full(64, 64), full(1, 64), full(64, 64), full(1, 64),
            full(128, 64), full(64, 64), full(64, 64), full(1, 64),
            full(64, 64), full(1, 64),
        ],
        out_specs=[
            pl.BlockSpec((bn, 64), lambda i: (i, 0)),
            pl.BlockSpec((bn, 3), lambda i: (i, 0)),
        ],
        out_shape=[
            jax.ShapeDtypeStruct((_N, 64), _f32),
            jax.ShapeDtypeStruct((_N, 3), _f32),
        ],
    )(h, xpad, acc0, acc1, acc2a, acc2b,
      wpn1, bpn1, wpn2, bpn2, wn1a, wn1b, wn1c, bn1, wn2, bn2)


def kernel(h, x, edge_index, W_fin, b_fin, W_f1, b_f1, W_f2, b_f2, W_sa,
           W_ew, b_ew, W_pn1, b_pn1, W_pn2, b_pn2, W_n1, b_n1, W_n2, b_n2,
           W_c1, b_c1, W_c2, b_c2):
    gs = jnp.concatenate([W_fin[0:128], W_ew[0:128]], axis=1)
    gd = jnp.concatenate([W_fin[128:256], W_ew[128:256]], axis=1)
    vs = jnp.linalg.solve(gs, W_sa[0:128])
    vd = jnp.linalg.solve(gd, W_sa[128:256])

    hp = jnp.pad(h, ((0, _NP - _N), (0, 0)))
    stab, dtab = _make_tables(hp, gs, gd)

    src = edge_index[0]
    dst = edge_index[1]
    x0 = x[:, 0]
    x1 = x[:, 1]
    x2 = x[:, 2]
    srow, drow, aux = _gather_rows(stab, dtab, x0, x1, x2, src, dst)

    ones64 = jnp.ones((1, 64), _f32)
    sel = (jnp.zeros((_W, 3), _f32)
           .at[0, 0].set(1.0).at[1, 0].set(1.0)
           .at[2, 1].set(1.0).at[3, 1].set(1.0)
           .at[4, 2].set(1.0).at[5, 2].set(1.0))
    p0 = sel[:, 0:1] * ones64
    p1 = sel[:, 1:2] * ones64
    p2 = sel[:, 2:3] * ones64
    o0, o1, o2 = _edge_compute(
        srow, drow, aux, p0, p1, p2, vs * ones64, vd * ones64,
        W_f1, b_f1[None, :], W_f2, b_f2[None, :],
        W_ew[256:320], b_ew[None, :], b_fin[None, :],
        W_c1, b_c1[None, :], W_c2 * ones64, b_c2[None, :])

    zrow = jnp.zeros((_NROW, _W), _f32)
    acc0, acc1, acc2a, acc2b = _scatter_rows(o0, o1, o2, dst, zrow)

    xpad = jnp.pad(x, ((0, 0), (0, 5)))
    h_new, x_new = _node_out(
        h, xpad, acc0, acc1, acc2a, acc2b,
        W_pn1, b_pn1[None, :], W_pn2, b_pn2[None, :],
        W_n1[0:128], W_n1[128:192], W_n1[192:256], b_n1[None, :],
        W_n2, b_n2[None, :])
    return h_new, x_new


# speedup vs baseline: 1.4275x; 1.0755x over previous
import functools

import jax
import jax.numpy as jnp
from jax import lax
from jax.experimental import pallas as pl
from jax.experimental.pallas import tpu as pltpu
from jax.experimental.pallas import tpu_sc as plsc

_N = 10000
_E = 320000
_IN_F = 128
_EPS = 1e-5
_NP = 10240
_W = 128

_f32 = jnp.float32

_info = plsc.get_sparse_core_info()
_NC = _info.num_cores
_NS = _info.num_subcores
_NW = _NC * _NS
_CH = 80
_EHALF = _E // 2
_EW = _EHALF // _NW
_NCHG = (_EW + _CH - 1) // _CH
_EC = _EHALF // _NS
_NROW = _NP // _NS

_mesh = plsc.VectorSubcoreMesh(core_axis_name="c", subcore_axis_name="s")


def _tab_body(h_ref, gs_ref, gd_ref, s_ref, d_ref):
    h = h_ref[...]
    s_ref[...] = jnp.dot(h, gs_ref[...], preferred_element_type=_f32)
    d_ref[...] = jnp.dot(h, gd_ref[...], preferred_element_type=_f32)


def _make_tables(hp, gs, gd):
    bn = 2048
    return pl.pallas_call(
        _tab_body,
        grid=(_NP // bn,),
        in_specs=[
            pl.BlockSpec((bn, _W), lambda i: (i, 0)),
            pl.BlockSpec((_W, _W), lambda i: (0, 0)),
            pl.BlockSpec((_W, _W), lambda i: (0, 0)),
        ],
        out_specs=[
            pl.BlockSpec((bn, _W), lambda i: (i, 0)),
            pl.BlockSpec((bn, _W), lambda i: (i, 0)),
        ],
        out_shape=[
            jax.ShapeDtypeStruct((_NP, _W), _f32),
            jax.ShapeDtypeStruct((_NP, _W), _f32),
        ],
    )(hp, gs, gd)


@functools.partial(
    pl.kernel,
    mesh=_mesh,
    out_type=[
        jax.ShapeDtypeStruct((_EHALF, _W), _f32),
        jax.ShapeDtypeStruct((_EHALF, _W), _f32),
        jax.ShapeDtypeStruct((_EHALF, _W), _f32),
    ],
    scratch_types=[
        pltpu.VMEM((_N,), _f32),
        pltpu.VMEM((_N,), _f32),
        pltpu.VMEM((_N,), _f32),
        pltpu.VMEM((2, _CH), jnp.int32),
        pltpu.VMEM((2, _CH), jnp.int32),
        pltpu.VMEM((2, _CH, _W), _f32),
        pltpu.VMEM((2, _CH, _W), _f32),
        pltpu.VMEM((2, _CH, _W), _f32),
        pltpu.SemaphoreType.DMA,
        pltpu.SemaphoreType.DMA,
        pltpu.SemaphoreType.DMA,
        pltpu.SemaphoreType.DMA,
        pltpu.SemaphoreType.DMA,
        pltpu.SemaphoreType.DMA,
    ],
    compiler_params=pltpu.CompilerParams(needs_layout_passes=False),
)
def _gather_rows(stab, dtab, x0, x1, x2, sidx, didx,
                 orow_s, orow_d, oaux,
                 x0_t, x1_t, x2_t, iv_s2, iv_d2, rv_s2, rv_d2, bd2,
                 sem_i0, sem_i1, sem_g0, sem_g1, sem_w0, sem_w1):
    pltpu.sync_copy(x0, x0_t)
    pltpu.sync_copy(x1, x1_t)
    pltpu.sync_copy(x2, x2_t)
    wid = lax.axis_index("s") * _NC + lax.axis_index("c")
    base = wid * _EW
    nchunk = _NCHG
    npairs = (nchunk + 1) // 2
    sem_i = (sem_i0, sem_i1)
    sem_g = (sem_g0, sem_g1)
    sem_w = (sem_w0, sem_w1)

    zv = jnp.zeros((16,), _f32)

    def zbody(r, carry):
        for p in range(2):
            for j in range(_W // 16):
                bd2[p, r, pl.ds(j * 16, 16)] = zv
        return carry

    lax.fori_loop(0, _CH, zbody, 0)

    def cbase(k):
        return base + jnp.minimum(k * _CH, _EW - _CH)

    def pf_idx(k, p):
        cb = cbase(k)
        pltpu.async_copy(sidx.at[pl.ds(cb, _CH)], iv_s2.at[p], sem_i[p])
        pltpu.async_copy(didx.at[pl.ds(cb, _CH)], iv_d2.at[p], sem_i[p])

    def drain_idx(k, p):
        cb = cbase(k)
        pltpu.make_async_copy(sidx.at[pl.ds(cb, _CH)], iv_s2.at[p], sem_i[p]).wait()
        pltpu.make_async_copy(didx.at[pl.ds(cb, _CH)], iv_d2.at[p], sem_i[p]).wait()

    def start_gathers(p):
        pltpu.async_copy(stab.at[iv_s2.at[p]], rv_s2.at[p], sem_g[p])
        pltpu.async_copy(dtab.at[iv_d2.at[p]], rv_d2.at[p], sem_g[p])

    def wait_gathers(p):
        pltpu.make_async_copy(stab.at[iv_s2.at[p]], rv_s2.at[p], sem_g[p]).wait()
        pltpu.make_async_copy(dtab.at[iv_d2.at[p]], rv_d2.at[p], sem_g[p]).wait()

    def aux_compute(p):
        for g in range(_CH // 16):
            sl = pl.ds(g * 16, 16)
            isv = iv_s2[p, sl]
            idv = iv_d2[p, sl]
            rows = g * 16 + jnp.arange(16, dtype=jnp.int32)
            for comp, xt in ((0, x0_t), (1, x1_t), (2, x2_t)):
                dv = plsc.load_gather(xt, [isv]) - plsc.load_gather(xt, [idv])
                hi = plsc.bitcast(
                    plsc.bitcast(dv, jnp.uint32) & jnp.uint32(0xFFFF0000), _f32)
                lo = dv - hi
                plsc.store_scatter(
                    bd2.at[p], [rows, jnp.full((16,), 2 * comp, jnp.int32)], hi)
                plsc.store_scatter(
                    bd2.at[p], [rows, jnp.full((16,), 2 * comp + 1, jnp.int32)], lo)

    def start_wb(k, p):
        cb = cbase(k)
        pltpu.async_copy(rv_s2.at[p], orow_s.at[pl.ds(cb, _CH)], sem_w[p])
        pltpu.async_copy(rv_d2.at[p], orow_d.at[pl.ds(cb, _CH)], sem_w[p])
        pltpu.async_copy(bd2.at[p], oaux.at[pl.ds(cb, _CH)], sem_w[p])

    def wait_wb(k, p):
        cb = cbase(k)
        pltpu.make_async_copy(rv_s2.at[p], orow_s.at[pl.ds(cb, _CH)], sem_w[p]).wait()
        pltpu.make_async_copy(rv_d2.at[p], orow_d.at[pl.ds(cb, _CH)], sem_w[p]).wait()
        pltpu.make_async_copy(bd2.at[p], oaux.at[pl.ds(cb, _CH)], sem_w[p]).wait()

    pf_idx(0, 0)
    pf_idx(1, 1)

    def body(j, carry):
        c0 = 2 * j
        c1 = c0 + 1
        drain_idx(c0, 0)
        start_gathers(0)
        aux_compute(0)
        drain_idx(c1, 1)
        start_gathers(1)
        aux_compute(1)
        wait_gathers(0)
        start_wb(c0, 0)
        pf_idx(c0 + 2, 0)
        wait_gathers(1)
        start_wb(c1, 1)
        pf_idx(c1 + 2, 1)
        wait_wb(c0, 0)
        wait_wb(c1, 1)
        return carry

    lax.fori_loop(0, npairs, body, 0)
    drain_idx(2 * npairs, 0)
    drain_idx(2 * npairs + 1, 1)


def _edge_body(s_ref, d_ref, aux_ref, p0_ref, p1_ref, p2_ref, vsr_ref, vdr_ref,
               wf1_ref, bf1_ref, wf2_ref, bf2_ref, wew3_ref, bew_ref,
               bfin_ref, wc1_ref, bc1_ref, wc2r_ref, bc2_ref,
               o0_ref, o1_ref, o2_ref):
    s = s_ref[...]
    d = d_ref[...]
    aux = aux_ref[...]
    a = s[:, 0:64]
    cc = s[:, 64:128]
    b = d[:, 0:64]
    dd = d[:, 64:128]
    n = s.shape[0]

    dxb = jnp.dot(aux, p0_ref[...], preferred_element_type=_f32)
    dyb = jnp.dot(aux, p1_ref[...], preferred_element_type=_f32)
    dzb = jnp.dot(aux, p2_ref[...], preferred_element_type=_f32)
    d2 = dxb * dxb + dyb * dyb + dzb * dzb + _EPS
    r0 = lax.rsqrt(d2)
    inv = r0 * (1.5 - 0.5 * d2 * r0 * r0)
    dist = d2 * inv
    mu = (5.0 / 63.0) * lax.broadcasted_iota(jnp.int32, (1, 64), 1).astype(_f32)
    t = dist - mu
    rbf = jnp.exp(-10.0 * t * t)
    hf0 = (a + b + bfin_ref[...]) * rbf
    hf1 = jnp.dot(hf0, wf1_ref[...], preferred_element_type=_f32) + bf1_ref[...]
    hf = hf1 * jax.nn.sigmoid(hf1)
    he = jnp.dot(hf, wf2_ref[...], preferred_element_type=_f32) + bf2_ref[...]
    z = (jnp.dot(s, vsr_ref[...], preferred_element_type=_f32)
         + jnp.dot(d, vdr_ref[...], preferred_element_type=_f32))
    att = z * jax.nn.sigmoid(z)
    w = jnp.exp(att)
    ew = jnp.tanh(cc + dd + jnp.dot(he, wew3_ref[...], preferred_element_type=_f32) + bew_ref[...])
    c1 = jnp.dot(he, wc1_ref[...], preferred_element_type=_f32) + bc1_ref[...]
    c1 = c1 * jax.nn.sigmoid(c1)
    cw = jnp.dot(c1, wc2r_ref[...], preferred_element_type=_f32) + bc2_ref[...]
    o0_ref[...] = jnp.concatenate([w * he, ew * (dxb * inv)], axis=1)
    o1_ref[...] = jnp.concatenate([ew * (dyb * inv), ew * (dzb * inv)], axis=1)
    o2_ref[...] = jnp.concatenate(
        [w[:, 0:1], jnp.ones((n, 1), _f32),
         (cw * dxb)[:, 0:1], (cw * dyb)[:, 0:1], (cw * dzb)[:, 0:1],
         jnp.zeros((n, 123), _f32)], axis=1)


def _edge_compute(srow, drow, aux, p0, p1, p2, vsr, vdr,
                  wf1, bf1, wf2, bf2, wew3, bew, bfin, wc1, bc1, wc2r, bc2):
    be = 1600
    full = lambda r, c: pl.BlockSpec((r, c), lambda i: (0, 0))
    return pl.pallas_call(
        _edge_body,
        grid=(_EHALF // be,),
        in_specs=[
            pl.BlockSpec((be, _W), lambda i: (i, 0)),
            pl.BlockSpec((be, _W), lambda i: (i, 0)),
            pl.BlockSpec((be, _W), lambda i: (i, 0)),
            full(_W, 64), full(_W, 64), full(_W, 64),
            full(_W, 64), full(_W, 64),
            full(64, 64), full(1, 64), full(64, 64), full(1, 64),
            full(64, 64), full(1, 64), full(1, 64),
            full(64, 64), full(1, 64), full(64, 64), full(1, 1),
        ],
        out_specs=[
            pl.BlockSpec((be, _W), lambda i: (i, 0)),
            pl.BlockSpec((be, _W), lambda i: (i, 0)),
            pl.BlockSpec((be, _W), lambda i: (i, 0)),
        ],
        out_shape=[
            jax.ShapeDtypeStruct((_EHALF, _W), _f32),
            jax.ShapeDtypeStruct((_EHALF, _W), _f32),
            jax.ShapeDtypeStruct((_EHALF, _W), _f32),
        ],
    )(srow, drow, aux, p0, p1, p2, vsr, vdr,
      wf1, bf1, wf2, bf2, wew3, bew, bfin, wc1, bc1, wc2r, bc2)


@functools.partial(
    pl.kernel,
    mesh=_mesh,
    out_type=[
        jax.ShapeDtypeStruct((_NP, _W), _f32),
        jax.ShapeDtypeStruct((_NP, _W), _f32),
        jax.ShapeDtypeStruct((_NP, _W), _f32),
        jax.ShapeDtypeStruct((_NP, _W), _f32),
    ],
    scratch_types=[
        pltpu.VMEM((_CH,), jnp.int32),
        pltpu.VMEM((_CH,), jnp.int32),
        pltpu.VMEM((_CH, _W), _f32),
        pltpu.VMEM((_CH, _W), _f32),
        pltpu.VMEM_SHARED((_NP, _W), _f32),
        pltpu.SemaphoreType.DMA,
        pltpu.SemaphoreType.DMA,
        pltpu.SemaphoreType.DMA,
        pltpu.SemaphoreType.DMA,
    ],
)
def _scatter_rows(o0h1, o1h1, o2h1, o0h2, o1h2, o2h2, didx1, didx2, zrow,
                  acc0, acc1, acc2a, acc2b,
                  iv_a, iv_b, rv_a, rv_b, acc_sp, sem_a, sem_b, sem_sa, sem_sb):
    c = lax.axis_index("c")
    s = lax.axis_index("s")
    rb = s * _NROW

    def accumulate(didx, edge_ref, ebase, nchunk):
        def cbase(k):
            return ebase + jnp.minimum(k, nchunk - 1) * _CH

        def pf(k, iv, rv, sem):
            cb = cbase(k)
            pltpu.async_copy(didx.at[pl.ds(cb, _CH)], iv, sem)
            pltpu.async_copy(edge_ref.at[pl.ds(cb, _CH)], rv, sem)

        def drain_pf(k, iv, rv, sem):
            cb = cbase(k)
            pltpu.make_async_copy(didx.at[pl.ds(cb, _CH)], iv, sem).wait()
            pltpu.make_async_copy(edge_ref.at[pl.ds(cb, _CH)], rv, sem).wait()

        npairs = (nchunk + 1) // 2
        pf(0, iv_a, rv_a, sem_a)
        pf(1, iv_b, rv_b, sem_b)

        def body(j, carry):
            c0 = 2 * j
            c1 = c0 + 1
            drain_pf(c0, iv_a, rv_a, sem_a)
            sca = pltpu.async_copy(rv_a, acc_sp.at[iv_a], sem_sa, add=True)
            drain_pf(c1, iv_b, rv_b, sem_b)

            @pl.when(c1 < nchunk)
            def _():
                pltpu.async_copy(rv_b, acc_sp.at[iv_b], sem_sb, add=True)

            sca.wait()
            pf(c0 + 2, iv_a, rv_a, sem_a)

            @pl.when(c1 < nchunk)
            def _():
                pltpu.make_async_copy(rv_b, acc_sp.at[iv_b], sem_sb).wait()

            pf(c1 + 2, iv_b, rv_b, sem_b)
            return carry

        lax.fori_loop(0, npairs, body, 0)
        drain_pf(2 * npairs, iv_a, rv_a, sem_a)
        drain_pf(2 * npairs + 1, iv_b, rv_b, sem_b)

    def flush(out_ref):
        pltpu.sync_copy(acc_sp.at[pl.ds(rb, _NROW)], out_ref.at[pl.ds(rb, _NROW)])

    pltpu.sync_copy(zrow, acc_sp.at[pl.ds(rb, _NROW)])
    plsc.subcore_barrier()

    @pl.when(c == 0)
    def _():
        accumulate(didx1, o0h1, s * _EC, _EC // _CH)
        accumulate(didx2, o0h2, s * _EC, _EC // _CH)

    @pl.when(c == 1)
    def _():
        accumulate(didx1, o1h1, s * _EC, _EC // _CH)
        accumulate(didx2, o1h2, s * _EC, _EC // _CH)

    plsc.subcore_barrier()

    @pl.when(c == 0)
    def _():
        flush(acc0)

    @pl.when(c == 1)
    def _():
        flush(acc1)

    plsc.subcore_barrier()

    pltpu.sync_copy(zrow, acc_sp.at[pl.ds(rb, _NROW)])
    plsc.subcore_barrier()

    @pl.when(c == 0)
    def _():
        accumulate(didx1, o2h1, s * _EC, _EC // _CH)

    @pl.when(c == 1)
    def _():
        accumulate(didx2, o2h2, s * _EC, _EC // _CH)

    plsc.subcore_barrier()

    @pl.when(c == 0)
    def _():
        flush(acc2a)

    @pl.when(c == 1)
    def _():
        flush(acc2b)


def _node_body(h_ref, x_ref, a0_ref, a1_ref, a2a_ref, a2b_ref,
               wpn1_ref, bpn1_ref, wpn2_ref, bpn2_ref,
               wn1a_ref, wn1b_ref, wn1c_ref, bn1_ref, wn2_ref, bn2_ref,
               hn_ref, xn_ref):
    a0 = a0_ref[...]
    a1 = a1_ref[...]
    a2 = a2a_ref[...] + a2b_ref[...]
    wsum = a2[:, 0:1]
    deg = a2[:, 1:2]
    cwd = a2[:, 2:5]
    heagg = a0[:, 0:64] / (wsum + _EPS)
    cx = a0[:, 64:128]
    cy = a1[:, 0:64]
    cz = a1[:, 64:128]
    cn = cx * cx + cy * cy + cz * cz
    t = jnp.dot(cn, wpn1_ref[...], preferred_element_type=_f32) + bpn1_ref[...]
    t = t * jax.nn.sigmoid(t)
    hcomb = jnp.dot(t, wpn2_ref[...], preferred_element_type=_f32) + bpn2_ref[...]
    h = h_ref[...]
    pre = (jnp.dot(h, wn1a_ref[...], preferred_element_type=_f32)
           + jnp.dot(heagg, wn1b_ref[...], preferred_element_type=_f32)
           + jnp.dot(hcomb, wn1c_ref[...], preferred_element_type=_f32)
           + bn1_ref[...])
    pre = pre * jax.nn.sigmoid(pre)
    hn_ref[...] = jnp.dot(pre, wn2_ref[...], preferred_element_type=_f32) + bn2_ref[...]
    xn_ref[...] = x_ref[...][:, 0:3] + cwd / (deg + 1.0)


def _node_out(h, xpad, acc0, acc1, acc2a, acc2b,
              wpn1, bpn1, wpn2, bpn2, wn1a, wn1b, wn1c, bn1, wn2, bn2):
    bn = 2000
    full = lambda r, c: pl.BlockSpec((r, c), lambda i: (0, 0))
    return pl.pallas_call(
        _node_body,
        grid=(_N // bn,),
        in_specs=[
            pl.BlockSpec((bn, _IN_F), lambda i: (i, 0)),
            pl.BlockSpec((bn, 8), lambda i: (i, 0)),
            pl.BlockSpec((bn, _W), lambda i: (i, 0)),
            pl.BlockSpec((bn, _W), lambda i: (i, 0)),
            pl.BlockSpec((bn, _W), lambda i: (i, 0)),
            pl.BlockSpec((bn, _W), lambda i: (i, 0)),
            full(64, 64), full(1, 64), full(64, 64), full(1, 64),
            full(128, 64), full(64, 64), full(64, 64), full(1, 64),
            full(64, 64), full(1, 64),
        ],
        out_specs=[
            pl.BlockSpec((bn, 64), lambda i: (i, 0)),
            pl.BlockSpec((bn, 3), lambda i: (i, 0)),
        ],
        out_shape=[
            jax.ShapeDtypeStruct((_N, 64), _f32),
            jax.ShapeDtypeStruct((_N, 3), _f32),
        ],
    )(h, xpad, acc0, acc1, acc2a, acc2b,
      wpn1, bpn1, wpn2, bpn2, wn1a, wn1b, wn1c, bn1, wn2, bn2)


def kernel(h, x, edge_index, W_fin, b_fin, W_f1, b_f1, W_f2, b_f2, W_sa,
           W_ew, b_ew, W_pn1, b_pn1, W_pn2, b_pn2, W_n1, b_n1, W_n2, b_n2,
           W_c1, b_c1, W_c2, b_c2):
    gs = jnp.concatenate([W_fin[0:128], W_ew[0:128]], axis=1)
    gd = jnp.concatenate([W_fin[128:256], W_ew[128:256]], axis=1)
    vs = jnp.linalg.solve(gs, W_sa[0:128])
    vd = jnp.linalg.solve(gd, W_sa[128:256])

    hp = jnp.pad(h, ((0, _NP - _N), (0, 0)))
    stab, dtab = _make_tables(hp, gs, gd)

    src = edge_index[0]
    dst = edge_index[1]
    x0 = x[:, 0]
    x1 = x[:, 1]
    x2 = x[:, 2]

    ones64 = jnp.ones((1, 64), _f32)
    sel = (jnp.zeros((_W, 3), _f32)
           .at[0, 0].set(1.0).at[1, 0].set(1.0)
           .at[2, 1].set(1.0).at[3, 1].set(1.0)
           .at[4, 2].set(1.0).at[5, 2].set(1.0))
    p0 = sel[:, 0:1] * ones64
    p1 = sel[:, 1:2] * ones64
    p2 = sel[:, 2:3] * ones64
    ew = (W_f1, b_f1[None, :], W_f2, b_f2[None, :],
          W_ew[256:320], b_ew[None, :], b_fin[None, :],
          W_c1, b_c1[None, :], W_c2 * ones64, b_c2[None, :])

    halves = []
    for k in range(2):
        sk = lax.dynamic_slice_in_dim(src, k * _EHALF, _EHALF)
        dk = lax.dynamic_slice_in_dim(dst, k * _EHALF, _EHALF)
        srow, drow, aux = _gather_rows(stab, dtab, x0, x1, x2, sk, dk)
        outs = _edge_compute(srow, drow, aux, p0, p1, p2,
                             vs * ones64, vd * ones64, *ew)
        halves.append((dk, outs))
    (d1, (o0h1, o1h1, o2h1)), (d2, (o0h2, o1h2, o2h2)) = halves

    zrow = jnp.zeros((_NROW, _W), _f32)
    acc0, acc1, acc2a, acc2b = _scatter_rows(
        o0h1, o1h1, o2h1, o0h2, o1h2, o2h2, d1, d2, zrow)

    xpad = jnp.pad(x, ((0, 0), (0, 5)))
    h_new, x_new = _node_out(
        h, xpad, acc0, acc1, acc2a, acc2b,
        W_pn1, b_pn1[None, :], W_pn2, b_pn2[None, :],
        W_n1[0:128], W_n1[128:192], W_n1[192:256], b_n1[None, :],
        W_n2, b_n2[None, :])
    return h_new, x_new


# speedup vs baseline: 1.5637x; 1.0954x over previous
import functools

import jax
import jax.numpy as jnp
from jax import lax
from jax.experimental import pallas as pl
from jax.experimental.pallas import tpu as pltpu
from jax.experimental.pallas import tpu_sc as plsc

_N = 10000
_E = 320000
_IN_F = 128
_EPS = 1e-5
_NP = 10240
_W = 128

_f32 = jnp.float32

_info = plsc.get_sparse_core_info()
_NC = _info.num_cores
_NS = _info.num_subcores
_NW = _NC * _NS
_CH = 80
_EHALF = _E // 2
_EW = _EHALF // _NW
_NCHG = (_EW + _CH - 1) // _CH
_EC = _EHALF // _NS
_NROW = _NP // _NS

_mesh = plsc.VectorSubcoreMesh(core_axis_name="c", subcore_axis_name="s")


def _tab_body(h_ref, gs_ref, gd_ref, s_ref, d_ref):
    h = h_ref[...]
    s_ref[...] = jnp.dot(h, gs_ref[...], preferred_element_type=_f32)
    d_ref[...] = jnp.dot(h, gd_ref[...], preferred_element_type=_f32)


def _make_tables(hp, gs, gd):
    bn = 2048
    return pl.pallas_call(
        _tab_body,
        grid=(_NP // bn,),
        in_specs=[
            pl.BlockSpec((bn, _W), lambda i: (i, 0)),
            pl.BlockSpec((_W, _W), lambda i: (0, 0)),
            pl.BlockSpec((_W, _W), lambda i: (0, 0)),
        ],
        out_specs=[
            pl.BlockSpec((bn, _W), lambda i: (i, 0)),
            pl.BlockSpec((bn, _W), lambda i: (i, 0)),
        ],
        out_shape=[
            jax.ShapeDtypeStruct((_NP, _W), _f32),
            jax.ShapeDtypeStruct((_NP, _W), _f32),
        ],
    )(hp, gs, gd)


@functools.partial(
    pl.kernel,
    mesh=_mesh,
    out_type=[
        jax.ShapeDtypeStruct((_EHALF, _W), _f32),
        jax.ShapeDtypeStruct((_EHALF, _W), _f32),
        jax.ShapeDtypeStruct((_EHALF, _W), _f32),
    ],
    scratch_types=[
        pltpu.VMEM((_N,), _f32),
        pltpu.VMEM((_N,), _f32),
        pltpu.VMEM((_N,), _f32),
        pltpu.VMEM((2, _CH), jnp.int32),
        pltpu.VMEM((2, _CH), jnp.int32),
        pltpu.VMEM((2, _CH, _W), _f32),
        pltpu.VMEM((2, _CH, _W), _f32),
        pltpu.VMEM((2, _CH, _W), _f32),
        pltpu.SemaphoreType.DMA,
        pltpu.SemaphoreType.DMA,
        pltpu.SemaphoreType.DMA,
        pltpu.SemaphoreType.DMA,
        pltpu.SemaphoreType.DMA,
        pltpu.SemaphoreType.DMA,
    ],
    compiler_params=pltpu.CompilerParams(needs_layout_passes=False),
)
def _gather_rows(stab, dtab, x0, x1, x2, sidx, didx,
                 orow_s, orow_d, oaux,
                 x0_t, x1_t, x2_t, iv_s2, iv_d2, rv_s2, rv_d2, bd2,
                 sem_i0, sem_i1, sem_g0, sem_g1, sem_w0, sem_w1):
    pltpu.sync_copy(x0, x0_t)
    pltpu.sync_copy(x1, x1_t)
    pltpu.sync_copy(x2, x2_t)
    wid = lax.axis_index("s") * _NC + lax.axis_index("c")
    base = wid * _EW
    nchunk = _NCHG
    npairs = (nchunk + 1) // 2
    sem_i = (sem_i0, sem_i1)
    sem_g = (sem_g0, sem_g1)
    sem_w = (sem_w0, sem_w1)

    zv = jnp.zeros((16,), _f32)

    def zbody(r, carry):
        for p in range(2):
            for j in range(_W // 16):
                bd2[p, r, pl.ds(j * 16, 16)] = zv
        return carry

    lax.fori_loop(0, _CH, zbody, 0)

    def cbase(k):
        return base + jnp.minimum(k * _CH, _EW - _CH)

    def pf_idx(k, p):
        cb = cbase(k)
        pltpu.async_copy(sidx.at[pl.ds(cb, _CH)], iv_s2.at[p], sem_i[p])
        pltpu.async_copy(didx.at[pl.ds(cb, _CH)], iv_d2.at[p], sem_i[p])

    def drain_idx(k, p):
        cb = cbase(k)
        pltpu.make_async_copy(sidx.at[pl.ds(cb, _CH)], iv_s2.at[p], sem_i[p]).wait()
        pltpu.make_async_copy(didx.at[pl.ds(cb, _CH)], iv_d2.at[p], sem_i[p]).wait()

    def start_gathers(p):
        pltpu.async_copy(stab.at[iv_s2.at[p]], rv_s2.at[p], sem_g[p])
        pltpu.async_copy(dtab.at[iv_d2.at[p]], rv_d2.at[p], sem_g[p])

    def wait_gathers(p):
        pltpu.make_async_copy(stab.at[iv_s2.at[p]], rv_s2.at[p], sem_g[p]).wait()
        pltpu.make_async_copy(dtab.at[iv_d2.at[p]], rv_d2.at[p], sem_g[p]).wait()

    def aux_compute(p):
        for g in range(_CH // 16):
            sl = pl.ds(g * 16, 16)
            isv = iv_s2[p, sl]
            idv = iv_d2[p, sl]
            rows = g * 16 + jnp.arange(16, dtype=jnp.int32)
            for comp, xt in ((0, x0_t), (1, x1_t), (2, x2_t)):
                dv = plsc.load_gather(xt, [isv]) - plsc.load_gather(xt, [idv])
                hi = plsc.bitcast(
                    plsc.bitcast(dv, jnp.uint32) & jnp.uint32(0xFFFF0000), _f32)
                lo = dv - hi
                plsc.store_scatter(
                    bd2.at[p], [rows, jnp.full((16,), 2 * comp, jnp.int32)], hi)
                plsc.store_scatter(
                    bd2.at[p], [rows, jnp.full((16,), 2 * comp + 1, jnp.int32)], lo)

    def start_wb(k, p):
        cb = cbase(k)
        pltpu.async_copy(rv_s2.at[p], orow_s.at[pl.ds(cb, _CH)], sem_w[p])
        pltpu.async_copy(rv_d2.at[p], orow_d.at[pl.ds(cb, _CH)], sem_w[p])
        pltpu.async_copy(bd2.at[p], oaux.at[pl.ds(cb, _CH)], sem_w[p])

    def wait_wb(k, p):
        cb = cbase(k)
        pltpu.make_async_copy(rv_s2.at[p], orow_s.at[pl.ds(cb, _CH)], sem_w[p]).wait()
        pltpu.make_async_copy(rv_d2.at[p], orow_d.at[pl.ds(cb, _CH)], sem_w[p]).wait()
        pltpu.make_async_copy(bd2.at[p], oaux.at[pl.ds(cb, _CH)], sem_w[p]).wait()

    pf_idx(0, 0)
    pf_idx(1, 1)

    def body(j, carry):
        c0 = 2 * j
        c1 = c0 + 1
        drain_idx(c0, 0)
        start_gathers(0)
        aux_compute(0)
        drain_idx(c1, 1)
        start_gathers(1)
        aux_compute(1)
        wait_gathers(0)
        start_wb(c0, 0)
        pf_idx(c0 + 2, 0)
        wait_gathers(1)
        start_wb(c1, 1)
        pf_idx(c1 + 2, 1)
        wait_wb(c0, 0)
        wait_wb(c1, 1)
        return carry

    lax.fori_loop(0, npairs, body, 0)
    drain_idx(2 * npairs, 0)
    drain_idx(2 * npairs + 1, 1)


def _edge_body(s_ref, d_ref, aux_ref, p0_ref, p1_ref, p2_ref, vsr_ref, vdr_ref,
               wf1_ref, bf1_ref, wf2_ref, bf2_ref, wew3_ref, bew_ref,
               bfin_ref, wc1_ref, bc1_ref, wc2r_ref, bc2_ref,
               o0_ref, o1_ref, o2_ref):
    s = s_ref[...]
    d = d_ref[...]
    aux = aux_ref[...]
    a = s[:, 0:64]
    cc = s[:, 64:128]
    b = d[:, 0:64]
    dd = d[:, 64:128]
    n = s.shape[0]

    dxb = jnp.dot(aux, p0_ref[...], preferred_element_type=_f32)
    dyb = jnp.dot(aux, p1_ref[...], preferred_element_type=_f32)
    dzb = jnp.dot(aux, p2_ref[...], preferred_element_type=_f32)
    d2 = dxb * dxb + dyb * dyb + dzb * dzb + _EPS
    r0 = lax.rsqrt(d2)
    inv = r0 * (1.5 - 0.5 * d2 * r0 * r0)
    dist = d2 * inv
    mu = (5.0 / 63.0) * lax.broadcasted_iota(jnp.int32, (1, 64), 1).astype(_f32)
    t = dist - mu
    rbf = jnp.exp(-10.0 * t * t)
    hf0 = (a + b + bfin_ref[...]) * rbf
    hf1 = jnp.dot(hf0, wf1_ref[...], preferred_element_type=_f32) + bf1_ref[...]
    hf = hf1 * jax.nn.sigmoid(hf1)
    he = jnp.dot(hf, wf2_ref[...], preferred_element_type=_f32) + bf2_ref[...]
    z = (jnp.dot(s, vsr_ref[...], preferred_element_type=_f32)
         + jnp.dot(d, vdr_ref[...], preferred_element_type=_f32))
    att = z * jax.nn.sigmoid(z)
    w = jnp.exp(att)
    ew = jnp.tanh(cc + dd + jnp.dot(he, wew3_ref[...], preferred_element_type=_f32) + bew_ref[...])
    c1 = jnp.dot(he, wc1_ref[...], preferred_element_type=_f32) + bc1_ref[...]
    c1 = c1 * jax.nn.sigmoid(c1)
    cw = jnp.dot(c1, wc2r_ref[...], preferred_element_type=_f32) + bc2_ref[...]
    o0_ref[...] = jnp.concatenate([w * he, ew * (dxb * inv)], axis=1)
    o1_ref[...] = jnp.concatenate([ew * (dyb * inv), ew * (dzb * inv)], axis=1)
    o2_ref[...] = jnp.concatenate(
        [w[:, 0:1], jnp.ones((n, 1), _f32),
         (cw * dxb)[:, 0:1], (cw * dyb)[:, 0:1], (cw * dzb)[:, 0:1],
         jnp.zeros((n, 123), _f32)], axis=1)


def _edge_compute(srow, drow, aux, p0, p1, p2, vsr, vdr,
                  wf1, bf1, wf2, bf2, wew3, bew, bfin, wc1, bc1, wc2r, bc2):
    be = 1600
    full = lambda r, c: pl.BlockSpec((r, c), lambda i: (0, 0))
    return pl.pallas_call(
        _edge_body,
        grid=(_EHALF // be,),
        in_specs=[
            pl.BlockSpec((be, _W), lambda i: (i, 0)),
            pl.BlockSpec((be, _W), lambda i: (i, 0)),
            pl.BlockSpec((be, _W), lambda i: (i, 0)),
            full(_W, 64), full(_W, 64), full(_W, 64),
            full(_W, 64), full(_W, 64),
            full(64, 64), full(1, 64), full(64, 64), full(1, 64),
            full(64, 64), full(1, 64), full(1, 64),
            full(64, 64), full(1, 64), full(64, 64), full(1, 1),
        ],
        out_specs=[
            pl.BlockSpec((be, _W), lambda i: (i, 0)),
            pl.BlockSpec((be, _W), lambda i: (i, 0)),
            pl.BlockSpec((be, _W), lambda i: (i, 0)),
        ],
        out_shape=[
            jax.ShapeDtypeStruct((_EHALF, _W), _f32),
            jax.ShapeDtypeStruct((_EHALF, _W), _f32),
            jax.ShapeDtypeStruct((_EHALF, _W), _f32),
        ],
    )(srow, drow, aux, p0, p1, p2, vsr, vdr,
      wf1, bf1, wf2, bf2, wew3, bew, bfin, wc1, bc1, wc2r, bc2)


def _make_scatter(o2_core):
  @functools.partial(
      pl.kernel,
      mesh=_mesh,
      out_type=[
          jax.ShapeDtypeStruct((_NP, _W), _f32),
          jax.ShapeDtypeStruct((_NP, _W), _f32),
          jax.ShapeDtypeStruct((_NP, _W), _f32),
      ],
      scratch_types=[
          pltpu.VMEM((_CH,), jnp.int32),
          pltpu.VMEM((_CH,), jnp.int32),
          pltpu.VMEM((_CH, _W), _f32),
          pltpu.VMEM((_CH, _W), _f32),
          pltpu.VMEM_SHARED((_NP, _W), _f32),
          pltpu.SemaphoreType.DMA,
          pltpu.SemaphoreType.DMA,
          pltpu.SemaphoreType.DMA,
          pltpu.SemaphoreType.DMA,
      ],
  )
  def _scatter_half(o0, o1, o2, didx, zrow, acc0, acc1, acc2,
                    iv_a, iv_b, rv_a, rv_b, acc_sp,
                    sem_a, sem_b, sem_sa, sem_sb):
    c = lax.axis_index("c")
    s = lax.axis_index("s")
    rb = s * _NROW

    def accumulate(didx, edge_ref, ebase, nchunk):
        def cbase(k):
            return ebase + jnp.minimum(k, nchunk - 1) * _CH

        def pf(k, iv, rv, sem):
            cb = cbase(k)
            pltpu.async_copy(didx.at[pl.ds(cb, _CH)], iv, sem)
            pltpu.async_copy(edge_ref.at[pl.ds(cb, _CH)], rv, sem)

        def drain_pf(k, iv, rv, sem):
            cb = cbase(k)
            pltpu.make_async_copy(didx.at[pl.ds(cb, _CH)], iv, sem).wait()
            pltpu.make_async_copy(edge_ref.at[pl.ds(cb, _CH)], rv, sem).wait()

        npairs = (nchunk + 1) // 2
        pf(0, iv_a, rv_a, sem_a)
        pf(1, iv_b, rv_b, sem_b)

        def body(j, carry):
            c0 = 2 * j
            c1 = c0 + 1
            drain_pf(c0, iv_a, rv_a, sem_a)
            sca = pltpu.async_copy(rv_a, acc_sp.at[iv_a], sem_sa, add=True)
            drain_pf(c1, iv_b, rv_b, sem_b)

            @pl.when(c1 < nchunk)
            def _():
                pltpu.async_copy(rv_b, acc_sp.at[iv_b], sem_sb, add=True)

            sca.wait()
            pf(c0 + 2, iv_a, rv_a, sem_a)

            @pl.when(c1 < nchunk)
            def _():
                pltpu.make_async_copy(rv_b, acc_sp.at[iv_b], sem_sb).wait()

            pf(c1 + 2, iv_b, rv_b, sem_b)
            return carry

        lax.fori_loop(0, npairs, body, 0)
        drain_pf(2 * npairs, iv_a, rv_a, sem_a)
        drain_pf(2 * npairs + 1, iv_b, rv_b, sem_b)

    def flush(out_ref):
        pltpu.sync_copy(acc_sp.at[pl.ds(rb, _NROW)], out_ref.at[pl.ds(rb, _NROW)])

    pltpu.sync_copy(zrow, acc_sp.at[pl.ds(rb, _NROW)])
    plsc.subcore_barrier()

    @pl.when(c == 0)
    def _():
        accumulate(didx, o0, s * _EC, _EC // _CH)

    @pl.when(c == 1)
    def _():
        accumulate(didx, o1, s * _EC, _EC // _CH)

    plsc.subcore_barrier()

    @pl.when(c == 0)
    def _():
        flush(acc0)

    @pl.when(c == 1)
    def _():
        flush(acc1)

    plsc.subcore_barrier()

    pltpu.sync_copy(zrow, acc_sp.at[pl.ds(rb, _NROW)])
    plsc.subcore_barrier()

    @pl.when(c == o2_core)
    def _():
        accumulate(didx, o2, s * _EC, _EC // _CH)

    plsc.subcore_barrier()

    @pl.when(c == o2_core)
    def _():
        flush(acc2)

  return _scatter_half


_scatter_half_0 = _make_scatter(0)
_scatter_half_1 = _make_scatter(1)


def _node_body(h_ref, x_ref, a0a_ref, a1a_ref, a2a_ref,
               a0b_ref, a1b_ref, a2b_ref,
               wpn1_ref, bpn1_ref, wpn2_ref, bpn2_ref,
               wn1a_ref, wn1b_ref, wn1c_ref, bn1_ref, wn2_ref, bn2_ref,
               hn_ref, xn_ref):
    a0 = a0a_ref[...] + a0b_ref[...]
    a1 = a1a_ref[...] + a1b_ref[...]
    a2 = a2a_ref[...] + a2b_ref[...]
    wsum = a2[:, 0:1]
    deg = a2[:, 1:2]
    cwd = a2[:, 2:5]
    heagg = a0[:, 0:64] / (wsum + _EPS)
    cx = a0[:, 64:128]
    cy = a1[:, 0:64]
    cz = a1[:, 64:128]
    cn = cx * cx + cy * cy + cz * cz
    t = jnp.dot(cn, wpn1_ref[...], preferred_element_type=_f32) + bpn1_ref[...]
    t = t * jax.nn.sigmoid(t)
    hcomb = jnp.dot(t, wpn2_ref[...], preferred_element_type=_f32) + bpn2_ref[...]
    h = h_ref[...]
    pre = (jnp.dot(h, wn1a_ref[...], preferred_element_type=_f32)
           + jnp.dot(heagg, wn1b_ref[...], preferred_element_type=_f32)
           + jnp.dot(hcomb, wn1c_ref[...], preferred_element_type=_f32)
           + bn1_ref[...])
    pre = pre * jax.nn.sigmoid(pre)
    hn_ref[...] = jnp.dot(pre, wn2_ref[...], preferred_element_type=_f32) + bn2_ref[...]
    xn_ref[...] = x_ref[...][:, 0:3] + cwd / (deg + 1.0)


def _node_out(h, xpad, accs,
              wpn1, bpn1, wpn2, bpn2, wn1a, wn1b, wn1c, bn1, wn2, bn2):
    bn = 2000
    full = lambda r, c: pl.BlockSpec((r, c), lambda i: (0, 0))
    return pl.pallas_call(
        _node_body,
        grid=(_N // bn,),
        in_specs=[
            pl.BlockSpec((bn, _IN_F), lambda i: (i, 0)),
            pl.BlockSpec((bn, 8), lambda i: (i, 0)),
            pl.BlockSpec((bn, _W), lambda i: (i, 0)),
            pl.BlockSpec((bn, _W), lambda i: (i, 0)),
            pl.BlockSpec((bn, _W), lambda i: (i, 0)),
            pl.BlockSpec((bn, _W), lambda i: (i, 0)),
            pl.BlockSpec((bn, _W), lambda i: (i, 0)),
            pl.BlockSpec((bn, _W), lambda i: (i, 0)),
            full(64, 64), full(1, 64), full(64, 64), full(1, 64),
            full(128, 64), full(64, 64), full(64, 64), full(1, 64),
            full(64, 64), full(1, 64),
        ],
        out_specs=[
            pl.BlockSpec((bn, 64), lambda i: (i, 0)),
            pl.BlockSpec((bn, 3), lambda i: (i, 0)),
        ],
        out_shape=[
            jax.ShapeDtypeStruct((_N, 64), _f32),
            jax.ShapeDtypeStruct((_N, 3), _f32),
        ],
    )(h, xpad, *accs,
      wpn1, bpn1, wpn2, bpn2, wn1a, wn1b, wn1c, bn1, wn2, bn2)


def kernel(h, x, edge_index, W_fin, b_fin, W_f1, b_f1, W_f2, b_f2, W_sa,
           W_ew, b_ew, W_pn1, b_pn1, W_pn2, b_pn2, W_n1, b_n1, W_n2, b_n2,
           W_c1, b_c1, W_c2, b_c2):
    gs = jnp.concatenate([W_fin[0:128], W_ew[0:128]], axis=1)
    gd = jnp.concatenate([W_fin[128:256], W_ew[128:256]], axis=1)
    vs = jnp.linalg.solve(gs, W_sa[0:128])
    vd = jnp.linalg.solve(gd, W_sa[128:256])

    hp = jnp.pad(h, ((0, _NP - _N), (0, 0)))
    stab, dtab = _make_tables(hp, gs, gd)

    src = edge_index[0]
    dst = edge_index[1]
    x0 = x[:, 0]
    x1 = x[:, 1]
    x2 = x[:, 2]

    ones64 = jnp.ones((1, 64), _f32)
    sel = (jnp.zeros((_W, 3), _f32)
           .at[0, 0].set(1.0).at[1, 0].set(1.0)
           .at[2, 1].set(1.0).at[3, 1].set(1.0)
           .at[4, 2].set(1.0).at[5, 2].set(1.0))
    p0 = sel[:, 0:1] * ones64
    p1 = sel[:, 1:2] * ones64
    p2 = sel[:, 2:3] * ones64
    ew = (W_f1, b_f1[None, :], W_f2, b_f2[None, :],
          W_ew[256:320], b_ew[None, :], b_fin[None, :],
          W_c1, b_c1[None, :], W_c2 * ones64, b_c2[None, :])

    zrow = jnp.zeros((_NROW, _W), _f32)
    scatters = (_scatter_half_0, _scatter_half_1)
    accs = []
    for k in range(2):
        sk = lax.dynamic_slice_in_dim(src, k * _EHALF, _EHALF)
        dk = lax.dynamic_slice_in_dim(dst, k * _EHALF, _EHALF)
        srow, drow, aux = _gather_rows(stab, dtab, x0, x1, x2, sk, dk)
        o0, o1, o2 = _edge_compute(srow, drow, aux, p0, p1, p2,
                                   vs * ones64, vd * ones64, *ew)
        accs.extend(scatters[k](o0, o1, o2, dk, zrow))

    xpad = jnp.pad(x, ((0, 0), (0, 5)))
    h_new, x_new = _node_out(
        h, xpad, accs,
        W_pn1, b_pn1[None, :], W_pn2, b_pn2[None, :],
        W_n1[0:128], W_n1[128:192], W_n1[192:256], b_n1[None, :],
        W_n2, b_n2[None, :])
    return h_new, x_new


# speedup vs baseline: 1.5972x; 1.0214x over previous
import functools

import jax
import jax.numpy as jnp
from jax import lax
from jax.experimental import pallas as pl
from jax.experimental.pallas import tpu as pltpu
from jax.experimental.pallas import tpu_sc as plsc

_N = 10000
_E = 320000
_IN_F = 128
_EPS = 1e-5
_NP = 10240
_W = 128

_f32 = jnp.float32

_info = plsc.get_sparse_core_info()
_NC = _info.num_cores
_NS = _info.num_subcores
_NW = _NC * _NS
_CH = 80
_EHALF = _E // 2
_EW = _EHALF // _NW
_NCHG = (_EW + _CH - 1) // _CH
_EC = _EHALF // _NS
_NROW = _NP // _NS

_mesh = plsc.VectorSubcoreMesh(core_axis_name="c", subcore_axis_name="s")


def _tab_body(h_ref, gs_ref, gd_ref, s_ref, d_ref):
    h = h_ref[...]
    s_ref[...] = jnp.dot(h, gs_ref[...], preferred_element_type=_f32)
    d_ref[...] = jnp.dot(h, gd_ref[...], preferred_element_type=_f32)


def _make_tables(hp, gs, gd):
    bn = 2048
    return pl.pallas_call(
        _tab_body,
        grid=(_NP // bn,),
        in_specs=[
            pl.BlockSpec((bn, _W), lambda i: (i, 0)),
            pl.BlockSpec((_W, _W), lambda i: (0, 0)),
            pl.BlockSpec((_W, _W), lambda i: (0, 0)),
        ],
        out_specs=[
            pl.BlockSpec((bn, _W), lambda i: (i, 0)),
            pl.BlockSpec((bn, _W), lambda i: (i, 0)),
        ],
        out_shape=[
            jax.ShapeDtypeStruct((_NP, _W), _f32),
            jax.ShapeDtypeStruct((_NP, _W), _f32),
        ],
    )(hp, gs, gd)


@functools.partial(
    pl.kernel,
    mesh=_mesh,
    out_type=[
        jax.ShapeDtypeStruct((_EHALF, _W), _f32),
        jax.ShapeDtypeStruct((_EHALF, _W), _f32),
        jax.ShapeDtypeStruct((_EHALF, _W), _f32),
    ],
    scratch_types=[
        pltpu.VMEM((_N,), _f32),
        pltpu.VMEM((_N,), _f32),
        pltpu.VMEM((_N,), _f32),
        pltpu.VMEM((2, _CH), jnp.int32),
        pltpu.VMEM((2, _CH), jnp.int32),
        pltpu.VMEM((2, _CH, _W), _f32),
        pltpu.VMEM((2, _CH, _W), _f32),
        pltpu.VMEM((2, _CH, _W), _f32),
        pltpu.SemaphoreType.DMA,
        pltpu.SemaphoreType.DMA,
        pltpu.SemaphoreType.DMA,
        pltpu.SemaphoreType.DMA,
        pltpu.SemaphoreType.DMA,
        pltpu.SemaphoreType.DMA,
    ],
    compiler_params=pltpu.CompilerParams(needs_layout_passes=False),
)
def _gather_rows(stab, dtab, x0, x1, x2, sidx, didx,
                 orow_s, orow_d, oaux,
                 x0_t, x1_t, x2_t, iv_s2, iv_d2, rv_s2, rv_d2, bd2,
                 sem_i0, sem_i1, sem_g0, sem_g1, sem_w0, sem_w1):
    pltpu.sync_copy(x0, x0_t)
    pltpu.sync_copy(x1, x1_t)
    pltpu.sync_copy(x2, x2_t)
    wid = lax.axis_index("s") * _NC + lax.axis_index("c")
    base = wid * _EW
    nchunk = _NCHG
    npairs = (nchunk + 1) // 2
    sem_i = (sem_i0, sem_i1)
    sem_g = (sem_g0, sem_g1)
    sem_w = (sem_w0, sem_w1)

    zv = jnp.zeros((16,), _f32)

    def zbody(r, carry):
        for p in range(2):
            for j in range(_W // 16):
                bd2[p, r, pl.ds(j * 16, 16)] = zv
        return carry

    lax.fori_loop(0, _CH, zbody, 0)

    def cbase(k):
        return base + jnp.minimum(k * _CH, _EW - _CH)

    def pf_idx(k, p):
        cb = cbase(k)
        pltpu.async_copy(sidx.at[pl.ds(cb, _CH)], iv_s2.at[p], sem_i[p])
        pltpu.async_copy(didx.at[pl.ds(cb, _CH)], iv_d2.at[p], sem_i[p])

    def drain_idx(k, p):
        cb = cbase(k)
        pltpu.make_async_copy(sidx.at[pl.ds(cb, _CH)], iv_s2.at[p], sem_i[p]).wait()
        pltpu.make_async_copy(didx.at[pl.ds(cb, _CH)], iv_d2.at[p], sem_i[p]).wait()

    def start_gathers(p):
        pltpu.async_copy(stab.at[iv_s2.at[p]], rv_s2.at[p], sem_g[p])
        pltpu.async_copy(dtab.at[iv_d2.at[p]], rv_d2.at[p], sem_g[p])

    def wait_gathers(p):
        pltpu.make_async_copy(stab.at[iv_s2.at[p]], rv_s2.at[p], sem_g[p]).wait()
        pltpu.make_async_copy(dtab.at[iv_d2.at[p]], rv_d2.at[p], sem_g[p]).wait()

    def aux_compute(p):
        for g in range(_CH // 16):
            sl = pl.ds(g * 16, 16)
            isv = iv_s2[p, sl]
            idv = iv_d2[p, sl]
            rows = g * 16 + jnp.arange(16, dtype=jnp.int32)
            for comp, xt in ((0, x0_t), (1, x1_t), (2, x2_t)):
                dv = plsc.load_gather(xt, [isv]) - plsc.load_gather(xt, [idv])
                hi = plsc.bitcast(
                    plsc.bitcast(dv, jnp.uint32) & jnp.uint32(0xFFFF0000), _f32)
                lo = dv - hi
                plsc.store_scatter(
                    bd2.at[p], [rows, jnp.full((16,), 2 * comp, jnp.int32)], hi)
                plsc.store_scatter(
                    bd2.at[p], [rows, jnp.full((16,), 2 * comp + 1, jnp.int32)], lo)

    def start_wb(k, p):
        cb = cbase(k)
        pltpu.async_copy(rv_s2.at[p], orow_s.at[pl.ds(cb, _CH)], sem_w[p])
        pltpu.async_copy(rv_d2.at[p], orow_d.at[pl.ds(cb, _CH)], sem_w[p])
        pltpu.async_copy(bd2.at[p], oaux.at[pl.ds(cb, _CH)], sem_w[p])

    def wait_wb(k, p):
        cb = cbase(k)
        pltpu.make_async_copy(rv_s2.at[p], orow_s.at[pl.ds(cb, _CH)], sem_w[p]).wait()
        pltpu.make_async_copy(rv_d2.at[p], orow_d.at[pl.ds(cb, _CH)], sem_w[p]).wait()
        pltpu.make_async_copy(bd2.at[p], oaux.at[pl.ds(cb, _CH)], sem_w[p]).wait()

    pf_idx(0, 0)
    pf_idx(1, 1)

    def body(j, carry):
        c0 = 2 * j
        c1 = c0 + 1
        drain_idx(c0, 0)
        start_gathers(0)
        aux_compute(0)
        drain_idx(c1, 1)
        start_gathers(1)
        aux_compute(1)
        wait_gathers(0)
        start_wb(c0, 0)
        pf_idx(c0 + 2, 0)
        wait_gathers(1)
        start_wb(c1, 1)
        pf_idx(c1 + 2, 1)
        wait_wb(c0, 0)
        wait_wb(c1, 1)
        return carry

    lax.fori_loop(0, npairs, body, 0)
    drain_idx(2 * npairs, 0)
    drain_idx(2 * npairs + 1, 1)


def _edge_body(s_ref, d_ref, aux_ref, p0_ref, p1_ref, p2_ref, vsr_ref, vdr_ref,
               wf1_ref, bf1_ref, wf2_ref, bf2_ref, wew3_ref, bew_ref,
               bfin_ref, wc1_ref, bc1_ref, wc2r_ref, bc2_ref,
               o0_ref, o1_ref, o2_ref):
    s = s_ref[...]
    d = d_ref[...]
    aux = aux_ref[...]
    a = s[:, 0:64]
    cc = s[:, 64:128]
    b = d[:, 0:64]
    dd = d[:, 64:128]
    n = s.shape[0]

    dxb = jnp.dot(aux, p0_ref[...], preferred_element_type=_f32)
    dyb = jnp.dot(aux, p1_ref[...], preferred_element_type=_f32)
    dzb = jnp.dot(aux, p2_ref[...], preferred_element_type=_f32)
    d2 = dxb * dxb + dyb * dyb + dzb * dzb + _EPS
    r0 = lax.rsqrt(d2)
    inv = r0 * (1.5 - 0.5 * d2 * r0 * r0)
    dist = d2 * inv
    mu = (5.0 / 63.0) * lax.broadcasted_iota(jnp.int32, (1, 64), 1).astype(_f32)
    t = dist - mu
    rbf = jnp.exp(-10.0 * t * t)
    hf0 = (a + b + bfin_ref[...]) * rbf
    hf1 = jnp.dot(hf0, wf1_ref[...], preferred_element_type=_f32) + bf1_ref[...]
    hf = hf1 * jax.nn.sigmoid(hf1)
    he = jnp.dot(hf, wf2_ref[...], preferred_element_type=_f32) + bf2_ref[...]
    z = (jnp.dot(s, vsr_ref[...], preferred_element_type=_f32)
         + jnp.dot(d, vdr_ref[...], preferred_element_type=_f32))
    att = z * jax.nn.sigmoid(z)
    w = jnp.exp(att)
    ew = jnp.tanh(cc + dd + jnp.dot(he, wew3_ref[...], preferred_element_type=_f32) + bew_ref[...])
    c1 = jnp.dot(he, wc1_ref[...], preferred_element_type=_f32) + bc1_ref[...]
    c1 = c1 * jax.nn.sigmoid(c1)
    cw = jnp.dot(c1, wc2r_ref[...], preferred_element_type=_f32) + bc2_ref[...]
    o0_ref[...] = jnp.concatenate([w * he, ew * (dxb * inv)], axis=1)
    o1_ref[...] = jnp.concatenate([ew * (dyb * inv), ew * (dzb * inv)], axis=1)
    o2_ref[...] = jnp.concatenate(
        [w[:, 0:1], jnp.ones((n, 1), _f32),
         (cw * dxb)[:, 0:1], (cw * dyb)[:, 0:1], (cw * dzb)[:, 0:1],
         jnp.zeros((n, 123), _f32)], axis=1)


def _edge_compute(srow, drow, aux, p0, p1, p2, vsr, vdr,
                  wf1, bf1, wf2, bf2, wew3, bew, bfin, wc1, bc1, wc2r, bc2):
    be = 3200
    full = lambda r, c: pl.BlockSpec((r, c), lambda i: (0, 0))
    return pl.pallas_call(
        _edge_body,
        grid=(_EHALF // be,),
        in_specs=[
            pl.BlockSpec((be, _W), lambda i: (i, 0)),
            pl.BlockSpec((be, _W), lambda i: (i, 0)),
            pl.BlockSpec((be, _W), lambda i: (i, 0)),
            full(_W, 64), full(_W, 64), full(_W, 64),
            full(_W, 64), full(_W, 64),
            full(64, 64), full(1, 64), full(64, 64), full(1, 64),
            full(64, 64), full(1, 64), full(1, 64),
            full(64, 64), full(1, 64), full(64, 64), full(1, 1),
        ],
        out_specs=[
            pl.BlockSpec((be, _W), lambda i: (i, 0)),
            pl.BlockSpec((be, _W), lambda i: (i, 0)),
            pl.BlockSpec((be, _W), lambda i: (i, 0)),
        ],
        out_shape=[
            jax.ShapeDtypeStruct((_EHALF, _W), _f32),
            jax.ShapeDtypeStruct((_EHALF, _W), _f32),
            jax.ShapeDtypeStruct((_EHALF, _W), _f32),
        ],
    )(srow, drow, aux, p0, p1, p2, vsr, vdr,
      wf1, bf1, wf2, bf2, wew3, bew, bfin, wc1, bc1, wc2r, bc2)


def _make_scatter(o2_core):
  @functools.partial(
      pl.kernel,
      mesh=_mesh,
      out_type=[
          jax.ShapeDtypeStruct((_NP, _W), _f32),
          jax.ShapeDtypeStruct((_NP, _W), _f32),
          jax.ShapeDtypeStruct((_NP, _W), _f32),
      ],
      scratch_types=[
          pltpu.VMEM((_CH,), jnp.int32),
          pltpu.VMEM((_CH,), jnp.int32),
          pltpu.VMEM((_CH, _W), _f32),
          pltpu.VMEM((_CH, _W), _f32),
          pltpu.VMEM_SHARED((_NP, _W), _f32),
          pltpu.SemaphoreType.DMA,
          pltpu.SemaphoreType.DMA,
          pltpu.SemaphoreType.DMA,
          pltpu.SemaphoreType.DMA,
      ],
  )
  def _scatter_half(o0, o1, o2, didx, zrow, acc0, acc1, acc2,
                    iv_a, iv_b, rv_a, rv_b, acc_sp,
                    sem_a, sem_b, sem_sa, sem_sb):
    c = lax.axis_index("c")
    s = lax.axis_index("s")
    rb = s * _NROW

    def accumulate(didx, edge_ref, ebase, nchunk):
        def cbase(k):
            return ebase + jnp.minimum(k, nchunk - 1) * _CH

        def pf(k, iv, rv, sem):
            cb = cbase(k)
            pltpu.async_copy(didx.at[pl.ds(cb, _CH)], iv, sem)
            pltpu.async_copy(edge_ref.at[pl.ds(cb, _CH)], rv, sem)

        def drain_pf(k, iv, rv, sem):
            cb = cbase(k)
            pltpu.make_async_copy(didx.at[pl.ds(cb, _CH)], iv, sem).wait()
            pltpu.make_async_copy(edge_ref.at[pl.ds(cb, _CH)], rv, sem).wait()

        npairs = (nchunk + 1) // 2
        pf(0, iv_a, rv_a, sem_a)
        pf(1, iv_b, rv_b, sem_b)

        def body(j, carry):
            c0 = 2 * j
            c1 = c0 + 1
            drain_pf(c0, iv_a, rv_a, sem_a)
            sca = pltpu.async_copy(rv_a, acc_sp.at[iv_a], sem_sa, add=True)
            drain_pf(c1, iv_b, rv_b, sem_b)

            @pl.when(c1 < nchunk)
            def _():
                pltpu.async_copy(rv_b, acc_sp.at[iv_b], sem_sb, add=True)

            sca.wait()
            pf(c0 + 2, iv_a, rv_a, sem_a)

            @pl.when(c1 < nchunk)
            def _():
                pltpu.make_async_copy(rv_b, acc_sp.at[iv_b], sem_sb).wait()

            pf(c1 + 2, iv_b, rv_b, sem_b)
            return carry

        lax.fori_loop(0, npairs, body, 0)
        drain_pf(2 * npairs, iv_a, rv_a, sem_a)
        drain_pf(2 * npairs + 1, iv_b, rv_b, sem_b)

    def flush(out_ref):
        pltpu.sync_copy(acc_sp.at[pl.ds(rb, _NROW)], out_ref.at[pl.ds(rb, _NROW)])

    pltpu.sync_copy(zrow, acc_sp.at[pl.ds(rb, _NROW)])
    plsc.subcore_barrier()

    @pl.when(c == 0)
    def _():
        accumulate(didx, o0, s * _EC, _EC // _CH)

    @pl.when(c == 1)
    def _():
        accumulate(didx, o1, s * _EC, _EC // _CH)

    plsc.subcore_barrier()

    @pl.when(c == 0)
    def _():
        flush(acc0)

    @pl.when(c == 1)
    def _():
        flush(acc1)

    plsc.subcore_barrier()

    pltpu.sync_copy(zrow, acc_sp.at[pl.ds(rb, _NROW)])
    plsc.subcore_barrier()

    @pl.when(c == o2_core)
    def _():
        accumulate(didx, o2, s * _EC, _EC // _CH)

    plsc.subcore_barrier()

    @pl.when(c == o2_core)
    def _():
        flush(acc2)

  return _scatter_half


_scatter_half_0 = _make_scatter(0)
_scatter_half_1 = _make_scatter(1)


def _node_body(h_ref, x_ref, a0a_ref, a1a_ref, a2a_ref,
               a0b_ref, a1b_ref, a2b_ref,
               wpn1_ref, bpn1_ref, wpn2_ref, bpn2_ref,
               wn1a_ref, wn1b_ref, wn1c_ref, bn1_ref, wn2_ref, bn2_ref,
               hn_ref, xn_ref):
    a0 = a0a_ref[...] + a0b_ref[...]
    a1 = a1a_ref[...] + a1b_ref[...]
    a2 = a2a_ref[...] + a2b_ref[...]
    wsum = a2[:, 0:1]
    deg = a2[:, 1:2]
    cwd = a2[:, 2:5]
    heagg = a0[:, 0:64] / (wsum + _EPS)
    cx = a0[:, 64:128]
    cy = a1[:, 0:64]
    cz = a1[:, 64:128]
    cn = cx * cx + cy * cy + cz * cz
    t = jnp.dot(cn, wpn1_ref[...], preferred_element_type=_f32) + bpn1_ref[...]
    t = t * jax.nn.sigmoid(t)
    hcomb = jnp.dot(t, wpn2_ref[...], preferred_element_type=_f32) + bpn2_ref[...]
    h = h_ref[...]
    pre = (jnp.dot(h, wn1a_ref[...], preferred_element_type=_f32)
           + jnp.dot(heagg, wn1b_ref[...], preferred_element_type=_f32)
           + jnp.dot(hcomb, wn1c_ref[...], preferred_element_type=_f32)
           + bn1_ref[...])
    pre = pre * jax.nn.sigmoid(pre)
    hn_ref[...] = jnp.dot(pre, wn2_ref[...], preferred_element_type=_f32) + bn2_ref[...]
    xn_ref[...] = x_ref[...][:, 0:3] + cwd / (deg + 1.0)


def _node_out(h, xpad, accs,
              wpn1, bpn1, wpn2, bpn2, wn1a, wn1b, wn1c, bn1, wn2, bn2):
    bn = 2000
    full = lambda r, c: pl.BlockSpec((r, c), lambda i: (0, 0))
    return pl.pallas_call(
        _node_body,
        grid=(_N // bn,),
        in_specs=[
            pl.BlockSpec((bn, _IN_F), lambda i: (i, 0)),
            pl.BlockSpec((bn, 8), lambda i: (i, 0)),
            pl.BlockSpec((bn, _W), lambda i: (i, 0)),
            pl.BlockSpec((bn, _W), lambda i: (i, 0)),
            pl.BlockSpec((bn, _W), lambda i: (i, 0)),
            pl.BlockSpec((bn, _W), lambda i: (i, 0)),
            pl.BlockSpec((bn, _W), lambda i: (i, 0)),
            pl.BlockSpec((bn, _W), lambda i: (i, 0)),
            full(64, 64), full(1, 64), full(64, 64), full(1, 64),
            full(128, 64), full(64, 64), full(64, 64), full(1, 64),
            full(64, 64), full(1, 64),
        ],
        out_specs=[
            pl.BlockSpec((bn, 64), lambda i: (i, 0)),
            pl.BlockSpec((bn, 3), lambda i: (i, 0)),
        ],
        out_shape=[
            jax.ShapeDtypeStruct((_N, 64), _f32),
            jax.ShapeDtypeStruct((_N, 3), _f32),
        ],
    )(h, xpad, *accs,
      wpn1, bpn1, wpn2, bpn2, wn1a, wn1b, wn1c, bn1, wn2, bn2)


def kernel(h, x, edge_index, W_fin, b_fin, W_f1, b_f1, W_f2, b_f2, W_sa,
           W_ew, b_ew, W_pn1, b_pn1, W_pn2, b_pn2, W_n1, b_n1, W_n2, b_n2,
           W_c1, b_c1, W_c2, b_c2):
    gs = jnp.concatenate([W_fin[0:128], W_ew[0:128]], axis=1)
    gd = jnp.concatenate([W_fin[128:256], W_ew[128:256]], axis=1)
    vs = jnp.linalg.solve(gs, W_sa[0:128])
    vd = jnp.linalg.solve(gd, W_sa[128:256])

    hp = jnp.pad(h, ((0, _NP - _N), (0, 0)))
    stab, dtab = _make_tables(hp, gs, gd)

    src = edge_index[0]
    dst = edge_index[1]
    x0 = x[:, 0]
    x1 = x[:, 1]
    x2 = x[:, 2]

    ones64 = jnp.ones((1, 64), _f32)
    sel = (jnp.zeros((_W, 3), _f32)
           .at[0, 0].set(1.0).at[1, 0].set(1.0)
           .at[2, 1].set(1.0).at[3, 1].set(1.0)
           .at[4, 2].set(1.0).at[5, 2].set(1.0))
    p0 = sel[:, 0:1] * ones64
    p1 = sel[:, 1:2] * ones64
    p2 = sel[:, 2:3] * ones64
    ew = (W_f1, b_f1[None, :], W_f2, b_f2[None, :],
          W_ew[256:320], b_ew[None, :], b_fin[None, :],
          W_c1, b_c1[None, :], W_c2 * ones64, b_c2[None, :])

    zrow = jnp.zeros((_NROW, _W), _f32)
    scatters = (_scatter_half_0, _scatter_half_1)
    accs = []
    for k in range(2):
        sk = lax.dynamic_slice_in_dim(src, k * _EHALF, _EHALF)
        dk = lax.dynamic_slice_in_dim(dst, k * _EHALF, _EHALF)
        srow, drow, aux = _gather_rows(stab, dtab, x0, x1, x2, sk, dk)
        o0, o1, o2 = _edge_compute(srow, drow, aux, p0, p1, p2,
                                   vs * ones64, vd * ones64, *ew)
        accs.extend(scatters[k](o0, o1, o2, dk, zrow))

    xpad = jnp.pad(x, ((0, 0), (0, 5)))
    h_new, x_new = _node_out(
        h, xpad, accs,
        W_pn1, b_pn1[None, :], W_pn2, b_pn2[None, :],
        W_n1[0:128], W_n1[128:192], W_n1[192:256], b_n1[None, :],
        W_n2, b_n2[None, :])
    return h_new, x_new
